# Initial kernel scaffold; baseline (speedup 1.0000x reference)
import jax
import jax.numpy as jnp
from jax.experimental import pallas as pl


def kernel(pos, actor_type, lane_index, direction, params, edge_index, batch):
    raise NotImplementedError("write your pallas kernel here")


# speedup vs baseline: 4.4861x; 4.4861x over previous
import functools

import jax
import jax.numpy as jnp
from jax import lax
from jax.experimental import pallas as pl
from jax.experimental.pallas import tpu as pltpu
from jax.experimental.pallas import tpu_sc as plsc

N = 12000
E = 192000
G = 2000
NP = 12288

NC, NS = 2, 16

_MESH = plsc.VectorSubcoreMesh(
    core_axis_name="c", subcore_axis_name="s", num_cores=NC, num_subcores=NS
)


ROWS_PER_SUB = NP // NS
EDGES_PER_TILE = E // NS
GCN_CHUNK = 120
GCN_NCHUNK = EDGES_PER_TILE // GCN_CHUNK


@functools.partial(
    pl.kernel,
    out_type=jax.ShapeDtypeStruct((2 * NP, 128), jnp.float32),
    mesh=_MESH,
    scratch_types=[
        pltpu.VMEM((GCN_CHUNK,), jnp.int32),
        pltpu.VMEM((GCN_CHUNK,), jnp.int32),
        pltpu.VMEM((GCN_CHUNK, 128), jnp.float32),
        pltpu.VMEM_SHARED((NP, 128), jnp.float32),
    ],
)
def _gcn_sc(table, src, dst, out, sidx_v, didx_v, rows_v, acc):
    core = lax.axis_index("c")
    sub = lax.axis_index("s")
    pltpu.sync_copy(
        table.at[pl.ds(core * NP + sub * ROWS_PER_SUB, ROWS_PER_SUB)],
        acc.at[pl.ds(sub * ROWS_PER_SUB, ROWS_PER_SUB)],
    )
    plsc.subcore_barrier()

    def step(k, carry):
        base = core * E + sub * EDGES_PER_TILE + k * GCN_CHUNK
        pltpu.sync_copy(src.at[pl.ds(base, GCN_CHUNK)], sidx_v)
        pltpu.sync_copy(
            dst.at[pl.ds(sub * EDGES_PER_TILE + k * GCN_CHUNK, GCN_CHUNK)],
            didx_v,
        )
        pltpu.sync_copy(table.at[sidx_v], rows_v)
        pltpu.sync_copy(rows_v, acc.at[didx_v], add=True)
        return carry

    lax.fori_loop(0, GCN_NCHUNK, step, 0)
    plsc.subcore_barrier()
    pltpu.sync_copy(
        acc.at[pl.ds(sub * ROWS_PER_SUB, ROWS_PER_SUB)],
        out.at[pl.ds(core * NP + sub * ROWS_PER_SUB, ROWS_PER_SUB)],
    )


POOL_ROWS = 2048
POOL_ITEMS = 12288
POOL_PER_TILE = POOL_ITEMS // NS
POOL_CHUNK = 96
POOL_NCHUNK = POOL_PER_TILE // POOL_CHUNK
POOL_RPS = POOL_ROWS // NS
POOL_SLABS = 12


@functools.partial(
    pl.kernel,
    out_type=jax.ShapeDtypeStruct((POOL_SLABS, POOL_ROWS, 128), jnp.float32),
    mesh=_MESH,
    scratch_types=[
        pltpu.VMEM((POOL_CHUNK,), jnp.int32),
        pltpu.VMEM((POOL_CHUNK,), jnp.int32),
        pltpu.VMEM((POOL_CHUNK, 128), jnp.float32),
        pltpu.VMEM_SHARED((POOL_ROWS, 128), jnp.float32),
    ],
)
def _pool_sc(table, src, dst, zinit, out, sidx_v, didx_v, rows_v, acc):
    core = lax.axis_index("c")
    sub = lax.axis_index("s")
    for p in range(POOL_SLABS // 2):
        h = core * (POOL_SLABS // 2) + p
        pltpu.sync_copy(
            zinit.at[pl.ds(sub * POOL_RPS, POOL_RPS)],
            acc.at[pl.ds(sub * POOL_RPS, POOL_RPS)],
        )
        plsc.subcore_barrier()

        def step(k, carry):
            base = sub * POOL_PER_TILE + k * POOL_CHUNK
            pltpu.sync_copy(
                src.at[pl.ds(h * POOL_ITEMS + base, POOL_CHUNK)], sidx_v
            )
            pltpu.sync_copy(dst.at[pl.ds(base, POOL_CHUNK)], didx_v)
            pltpu.sync_copy(table.at[sidx_v], rows_v)
            pltpu.sync_copy(rows_v, acc.at[didx_v], add=True)
            return carry

        lax.fori_loop(0, POOL_NCHUNK, step, 0)
        plsc.subcore_barrier()
        pltpu.sync_copy(
            acc.at[pl.ds(sub * POOL_RPS, POOL_RPS)],
            out.at[h, pl.ds(sub * POOL_RPS, POOL_RPS)],
        )
        plsc.subcore_barrier()


BM = 512
NB = (N + BM - 1) // BM
NBG = 4


def _finalize_deg(degfull):
    def body(d_ref, o_ref):
        o_ref[...] = lax.rsqrt(d_ref[...])

    return pl.pallas_call(
        body,
        out_shape=jax.ShapeDtypeStruct((NP, 128), jnp.float32),
        grid=(1,),
        in_specs=[pl.BlockSpec((NP, 128), lambda i: (0, 0))],
        out_specs=pl.BlockSpec((NP, 128), lambda i: (0, 0)),
    )(degfull)


def _enc_a1(x19, dis, wenc, benc, w1):
    def body(x_ref, dis_ref, we_ref, be_ref, w1_ref, n0_ref, y2_ref):
        n0 = jnp.maximum(
            jnp.dot(x_ref[...], we_ref[...], preferred_element_type=jnp.float32)
            + be_ref[...],
            0.0,
        )
        n0_ref[...] = n0
        d = dis_ref[:, 0:1]
        y = jnp.dot(n0, w1_ref[...], preferred_element_type=jnp.float32) * d
        y2_ref[0] = y[:, :128]
        y2_ref[1] = y[:, 128:]

    return pl.pallas_call(
        body,
        out_shape=(
            jax.ShapeDtypeStruct((N, 512), jnp.float32),
            jax.ShapeDtypeStruct((2, NP, 128), jnp.float32),
        ),
        grid=(NB,),
        in_specs=[
            pl.BlockSpec((BM, 19), lambda i: (i, 0)),
            pl.BlockSpec((BM, 128), lambda i: (i, 0)),
            pl.BlockSpec((19, 512), lambda i: (0, 0)),
            pl.BlockSpec((1, 512), lambda i: (0, 0)),
            pl.BlockSpec((512, 256), lambda i: (0, 0)),
        ],
        out_specs=(
            pl.BlockSpec((BM, 512), lambda i: (i, 0)),
            pl.BlockSpec((2, BM, 128), lambda i: (0, i, 0)),
        ),
    )(x19, dis, wenc, benc, w1)


def _gcn_post(acc2, dis, b, wn=None, want_h=False):
    def body(*refs):
        if wn is not None:
            a_ref, dis_ref, b_ref, wn_ref = refs[:4]
            orefs = refs[4:]
        else:
            a_ref, dis_ref, b_ref = refs[:3]
            orefs = refs[3:]
        d = dis_ref[:, 0:1]
        a = jnp.concatenate([a_ref[0], a_ref[1]], axis=1)
        h = jnp.maximum(a * d + b_ref[...], 0.0)
        i = 0
        if want_h:
            orefs[i][...] = h
            i += 1
        if wn is not None:
            y = jnp.dot(h, wn_ref[...], preferred_element_type=jnp.float32) * d
            orefs[i][0] = y[:, :128]
            orefs[i][1] = y[:, 128:]

    in_specs = [
        pl.BlockSpec((2, BM, 128), lambda i: (0, i, 0)),
        pl.BlockSpec((BM, 128), lambda i: (i, 0)),
        pl.BlockSpec((1, 256), lambda i: (0, 0)),
    ]
    args = [acc2, dis, b]
    if wn is not None:
        in_specs.append(pl.BlockSpec((256, 256), lambda i: (0, 0)))
        args.append(wn)
    out_shape, out_specs = [], []
    if want_h:
        out_shape.append(jax.ShapeDtypeStruct((N, 256), jnp.float32))
        out_specs.append(pl.BlockSpec((BM, 256), lambda i: (i, 0)))
    if wn is not None:
        out_shape.append(jax.ShapeDtypeStruct((2, NP, 128), jnp.float32))
        out_specs.append(pl.BlockSpec((2, BM, 128), lambda i: (0, i, 0)))
    return pl.pallas_call(
        body,
        out_shape=tuple(out_shape),
        grid=(NB,),
        in_specs=in_specs,
        out_specs=tuple(out_specs),
    )(*args)


def _latent(pooled, eps, aggw, aggb, muw, mub, varw, varb, dftw, dftb):
    def body(p_ref, e_ref, aw, ab, mw, mb, vw, vb, dw, db,
             mu_ref, lv_ref, dft_ref):
        cnt = p_ref[10][:, 0:1]
        g = jnp.concatenate(
            [p_ref[h] for h in range(10)], axis=1
        ) / jnp.maximum(cnt, 1.0)
        latent = jnp.dot(g, aw[...], preferred_element_type=jnp.float32) + ab[...]
        mu = jnp.dot(latent, mw[...], preferred_element_type=jnp.float32) + mb[...]
        lv = jnp.dot(latent, vw[...], preferred_element_type=jnp.float32) + vb[...]
        mu_ref[...] = mu
        lv_ref[...] = lv
        z = e_ref[...] * jnp.exp(0.5 * lv) + mu
        dft_ref[...] = jnp.dot(z, dw[...], preferred_element_type=jnp.float32) + db[...]

    return pl.pallas_call(
        body,
        out_shape=(
            jax.ShapeDtypeStruct((G, 256), jnp.float32),
            jax.ShapeDtypeStruct((G, 256), jnp.float32),
            jax.ShapeDtypeStruct((G, 1536), jnp.float32),
        ),
        grid=(NBG,),
        in_specs=[
            pl.BlockSpec((POOL_SLABS, BM, 128), lambda i: (0, i, 0)),
            pl.BlockSpec((BM, 256), lambda i: (i, 0)),
            pl.BlockSpec((1280, 256), lambda i: (0, 0)),
            pl.BlockSpec((1, 256), lambda i: (0, 0)),
            pl.BlockSpec((256, 256), lambda i: (0, 0)),
            pl.BlockSpec((1, 256), lambda i: (0, 0)),
            pl.BlockSpec((256, 256), lambda i: (0, 0)),
            pl.BlockSpec((1, 256), lambda i: (0, 0)),
            pl.BlockSpec((256, 1536), lambda i: (0, 0)),
            pl.BlockSpec((1, 1536), lambda i: (0, 0)),
        ],
        out_specs=(
            pl.BlockSpec((BM, 256), lambda i: (i, 0)),
            pl.BlockSpec((BM, 256), lambda i: (i, 0)),
            pl.BlockSpec((BM, 1536), lambda i: (i, 0)),
        ),
    )(pooled, eps, aggw, aggb, muw, mub, varw, varb, dftw, dftb)


def _dec_a(z6, dis, w):
    def body(z_ref, dis_ref, w_ref, y2_ref):
        d = dis_ref[:, 0:1]
        h = jnp.maximum(z_ref[...], 0.0)
        y = jnp.dot(h, w_ref[...], preferred_element_type=jnp.float32) * d
        y2_ref[0] = y[:, :128]
        y2_ref[1] = y[:, 128:]

    return pl.pallas_call(
        body,
        out_shape=jax.ShapeDtypeStruct((2, NP, 128), jnp.float32),
        grid=(NB,),
        in_specs=[
            pl.BlockSpec((BM, 256), lambda i: (i, 0)),
            pl.BlockSpec((BM, 128), lambda i: (i, 0)),
            pl.BlockSpec((256, 256), lambda i: (0, 0)),
        ],
        out_specs=pl.BlockSpec((2, BM, 128), lambda i: (0, i, 0)),
    )(z6, dis, w)


def _heads(acc2, dis, b3, w0cat, b0cat, bdp, b1p):
    def body(a_ref, dis_ref, b3_ref, w0_ref, b0_ref, bd_ref, b1_ref, o_ref):
        d = dis_ref[:, 0:1]
        a = jnp.concatenate([a_ref[0], a_ref[1]], axis=1)
        h3 = jnp.maximum(a * d + b3_ref[...], 0.0)
        hh = jnp.maximum(
            jnp.dot(h3, w0_ref[...], preferred_element_type=jnp.float32)
            + b0_ref[...],
            0.0,
        )
        o_ref[...] = (
            jnp.dot(hh, bd_ref[...], preferred_element_type=jnp.float32)
            + b1_ref[...]
        )

    return pl.pallas_call(
        body,
        out_shape=jax.ShapeDtypeStruct((N, 128), jnp.float32),
        grid=(NB,),
        in_specs=[
            pl.BlockSpec((2, BM, 128), lambda i: (0, i, 0)),
            pl.BlockSpec((BM, 128), lambda i: (i, 0)),
            pl.BlockSpec((1, 256), lambda i: (0, 0)),
            pl.BlockSpec((256, 1280), lambda i: (0, 0)),
            pl.BlockSpec((1, 1280), lambda i: (0, 0)),
            pl.BlockSpec((1280, 128), lambda i: (0, 0)),
            pl.BlockSpec((1, 128), lambda i: (0, 0)),
        ],
        out_specs=pl.BlockSpec((BM, 128), lambda i: (i, 0)),
    )(acc2, dis, b3, w0cat, b0cat, bdp, b1p)


def _gcn_layer(y2, src2, col, dis, b, wn=None, want_h=False):
    acc = _gcn_sc(y2.reshape(2 * NP, 128), src2, col)
    return _gcn_post(acc.reshape(2, NP, 128), dis, b, wn=wn, want_h=want_h)


def kernel(pos, actor_type, lane_index, direction, params, edge_index, batch):
    p = params
    f32 = jnp.float32
    row, col = edge_index[0], edge_index[1]
    src2 = jnp.concatenate([row, row + NP])

    wenc = jnp.zeros((19, 512), f32)
    wenc = wenc.at[0:2, 0:128].set(p["pos_W"])
    wenc = wenc.at[2:8, 128:256].set(p["type_W"])
    wenc = wenc.at[8:18, 256:384].set(p["lane_W"])
    wenc = wenc.at[18:19, 384:512].set(p["dir_W"])
    benc = jnp.concatenate(
        [p["pos_b"], p["type_b"], p["lane_b"], p["dir_b"]]
    ).reshape(1, 512)
    x19 = jnp.concatenate([pos, actor_type, lane_index, direction], axis=1)

    w0cat = jnp.concatenate(
        [p["px0_W"], p["py0_W"], p["at0_W"], p["dr0_W"], p["li0_W"]], axis=1
    )
    b0cat = jnp.concatenate(
        [p["px0_b"], p["py0_b"], p["at0_b"], p["dr0_b"], p["li0_b"]]
    ).reshape(1, 1280)
    bdp = jnp.zeros((1280, 128), f32)
    bdp = bdp.at[0:256, 0:1].set(p["px1_W"])
    bdp = bdp.at[256:512, 1:2].set(p["py1_W"])
    bdp = bdp.at[512:768, 2:8].set(p["at1_W"])
    bdp = bdp.at[768:1024, 8:9].set(p["dr1_W"])
    bdp = bdp.at[1024:1280, 9:19].set(p["li1_W"])
    b1p = jnp.zeros((128,), f32)
    b1p = b1p.at[0:1].set(p["px1_b"])
    b1p = b1p.at[1:2].set(p["py1_b"])
    b1p = b1p.at[2:8].set(p["at1_b"])
    b1p = b1p.at[8:9].set(p["dr1_b"])
    b1p = b1p.at[9:19].set(p["li1_b"])
    b1p = b1p.reshape(1, 128)

    eps = jax.random.normal(jax.random.key(42), (G, 256), dtype=f32)

    degfull = _gcn_sc(jnp.ones((2 * NP, 128), f32), src2, col)
    dis = _finalize_deg(degfull[:NP])

    n0, y2 = _enc_a1(x19, dis, wenc, benc, p["e1_W"])
    h1, y2 = _gcn_layer(y2, src2, col, dis, p["e1_b"].reshape(1, 256),
                        wn=p["e2_W"], want_h=True)
    h2, y2 = _gcn_layer(y2, src2, col, dis, p["e2_b"].reshape(1, 256),
                        wn=p["e3_W"], want_h=True)
    (h3,) = _gcn_layer(y2, src2, col, dis, p["e3_b"].reshape(1, 256),
                       wn=None, want_h=True)

    xcat = jnp.concatenate(
        [n0, h1, h2, h3, jnp.ones((N, 256), f32)], axis=1
    )
    ptab = jnp.concatenate(
        [xcat[:, h * 128:(h + 1) * 128] for h in range(POOL_SLABS)], axis=0
    )
    ar = jnp.arange(N, dtype=jnp.int32)
    pad_src = jnp.zeros((POOL_ITEMS - N,), jnp.int32)
    srcp_half = jnp.concatenate([ar, pad_src])
    srcp = jnp.concatenate([srcp_half + h * N for h in range(POOL_SLABS)])
    batch_pad = jnp.concatenate(
        [batch, jnp.full((POOL_ITEMS - N,), G, jnp.int32)]
    )
    pooled = _pool_sc(ptab, srcp, batch_pad, jnp.zeros((POOL_ROWS, 128), f32))

    mu, lv, dft = _latent(
        pooled, eps,
        p["agg_W"], p["agg_b"].reshape(1, 256),
        p["mu_W"], p["mu_b"].reshape(1, 256),
        p["var_W"], p["var_b"].reshape(1, 256),
        p["dft_W"], p["dft_b"].reshape(1, 1536),
    )

    y2 = _dec_a(dft.reshape(N, 256), dis, p["d1_W"])
    (y2,) = _gcn_layer(y2, src2, col, dis, p["d1_b"].reshape(1, 256),
                       wn=p["d2_W"], want_h=False)
    (y2,) = _gcn_layer(y2, src2, col, dis, p["d2_b"].reshape(1, 256),
                       wn=p["d3_W"], want_h=False)
    accd3 = _gcn_sc(y2.reshape(2 * NP, 128), src2, col).reshape(2, NP, 128)
    out = _heads(accd3, dis, p["d3_b"].reshape(1, 256), w0cat, b0cat, bdp, b1p)

    pos_out = out[:, 0:2]
    acttype = out[:, 2:8]
    direc = out[:, 8:9]
    laneidx = out[:, 9:19]
    return (pos_out, acttype, direc, laneidx, lv, mu)


# speedup vs baseline: 7.9841x; 1.7797x over previous
import functools

import jax
import jax.numpy as jnp
from jax import lax
from jax.experimental import pallas as pl
from jax.experimental.pallas import tpu as pltpu
from jax.experimental.pallas import tpu_sc as plsc

N = 12000
E = 192000
G = 2000
NP = 12288

NC, NS = 2, 16

_MESH = plsc.VectorSubcoreMesh(
    core_axis_name="c", subcore_axis_name="s", num_cores=NC, num_subcores=NS
)


ROWS_PER_SUB = NP // NS
EDGES_PER_TILE = E // NS
GCN_CHUNK = 80
GCN_BLK = 25
GCN_NBLK = 6


@functools.partial(
    pl.kernel,
    out_type=jax.ShapeDtypeStruct((2 * NP, 128), jnp.float32),
    mesh=_MESH,
    scratch_types=[
        pltpu.VMEM((GCN_BLK, GCN_CHUNK), jnp.int32),
        pltpu.VMEM((GCN_BLK, GCN_CHUNK), jnp.int32),
        pltpu.VMEM((2, GCN_CHUNK, 128), jnp.float32),
        pltpu.VMEM_SHARED((NP, 128), jnp.float32),
        pltpu.SemaphoreType.DMA((2,)),
        pltpu.SemaphoreType.DMA((2,)),
    ],
)
def _gcn_sc(table, src, dst, out, sidx2, didx2, rows2, acc, sem_g, sem_s):
    core = lax.axis_index("c")
    sub = lax.axis_index("s")
    pltpu.sync_copy(
        table.at[pl.ds(core * NP + sub * ROWS_PER_SUB, ROWS_PER_SUB)],
        acc.at[pl.ds(sub * ROWS_PER_SUB, ROWS_PER_SUB)],
    )
    plsc.subcore_barrier()

    for b in range(GCN_NBLK):
        pltpu.sync_copy(src.at[core, sub, b], sidx2)
        pltpu.sync_copy(dst.at[sub, b], didx2)
        pltpu.async_copy(table.at[sidx2.at[0]], rows2.at[0], sem_g.at[0])

        def step(k, carry):
            slot = lax.rem(k, 2)
            nslot = lax.rem(k + 1, 2)

            @pl.when(k > 0)
            def _wait_prev_scatter():
                pltpu.make_async_copy(
                    rows2.at[nslot], acc.at[didx2.at[k - 1]], sem_s.at[nslot]
                ).wait()

            @pl.when(k + 1 < GCN_BLK)
            def _fire_next_gather():
                pltpu.async_copy(
                    table.at[sidx2.at[k + 1]], rows2.at[nslot], sem_g.at[nslot]
                )

            pltpu.make_async_copy(
                table.at[sidx2.at[k]], rows2.at[slot], sem_g.at[slot]
            ).wait()
            pltpu.async_copy(
                rows2.at[slot], acc.at[didx2.at[k]], sem_s.at[slot], add=True
            )
            return carry

        lax.fori_loop(0, GCN_BLK, step, 0)
        pltpu.make_async_copy(
            rows2.at[(GCN_BLK - 1) % 2],
            acc.at[didx2.at[GCN_BLK - 1]],
            sem_s.at[(GCN_BLK - 1) % 2],
        ).wait()

    plsc.subcore_barrier()
    pltpu.sync_copy(
        acc.at[pl.ds(sub * ROWS_PER_SUB, ROWS_PER_SUB)],
        out.at[pl.ds(core * NP + sub * ROWS_PER_SUB, ROWS_PER_SUB)],
    )


DEG_CHUNK = 120
DEG_NCHUNK = (E // 2) // NS // DEG_CHUNK


@functools.partial(
    pl.kernel,
    out_type=jax.ShapeDtypeStruct((2, NP, 128), jnp.float32),
    mesh=_MESH,
    scratch_types=[
        pltpu.VMEM((DEG_NCHUNK, DEG_CHUNK), jnp.int32),
        pltpu.VMEM((DEG_CHUNK, 128), jnp.float32),
        pltpu.VMEM_SHARED((NP, 128), jnp.float32),
        pltpu.SemaphoreType.DMA((2,)),
    ],
)
def _deg_sc(dst, half_init, ones_t, out, didx2, ones_v, acc, sem_s):
    core = lax.axis_index("c")
    sub = lax.axis_index("s")
    pltpu.sync_copy(dst.at[core, sub], didx2)
    pltpu.sync_copy(ones_t, ones_v)
    pltpu.sync_copy(
        half_init, acc.at[pl.ds(sub * ROWS_PER_SUB, ROWS_PER_SUB)]
    )
    plsc.subcore_barrier()

    def step(k, carry):
        slot = lax.rem(k, 2)

        @pl.when(k > 1)
        def _wait_prev():
            pltpu.make_async_copy(
                ones_v, acc.at[didx2.at[k - 2]], sem_s.at[slot]
            ).wait()

        pltpu.async_copy(
            ones_v, acc.at[didx2.at[k]], sem_s.at[slot], add=True
        )
        return carry

    lax.fori_loop(0, DEG_NCHUNK, step, 0)
    for t in (DEG_NCHUNK - 2, DEG_NCHUNK - 1):
        pltpu.make_async_copy(
            ones_v, acc.at[didx2.at[t]], sem_s.at[t % 2]
        ).wait()
    plsc.subcore_barrier()
    pltpu.sync_copy(
        acc.at[pl.ds(sub * ROWS_PER_SUB, ROWS_PER_SUB)],
        out.at[core, pl.ds(sub * ROWS_PER_SUB, ROWS_PER_SUB)],
    )


POOL_ROWS = 2048
POOL_ITEMS = 12288
POOL_PER_TILE = POOL_ITEMS // NS
POOL_CHUNK = 96
POOL_NCHUNK = POOL_PER_TILE // POOL_CHUNK
POOL_RPS = POOL_ROWS // NS
POOL_SLABS = 12


@functools.partial(
    pl.kernel,
    out_type=jax.ShapeDtypeStruct((POOL_SLABS, POOL_ROWS, 128), jnp.float32),
    mesh=_MESH,
    scratch_types=[
        pltpu.VMEM((POOL_NCHUNK, POOL_CHUNK), jnp.int32),
        pltpu.VMEM((POOL_NCHUNK, POOL_CHUNK), jnp.int32),
        pltpu.VMEM((2, POOL_CHUNK, 128), jnp.float32),
        pltpu.VMEM_SHARED((POOL_ROWS, 128), jnp.float32),
        pltpu.SemaphoreType.DMA((2,)),
        pltpu.SemaphoreType.DMA((2,)),
    ],
)
def _pool_sc(table, src, dst, zinit, out, sidx2, didx2, rows2, acc, sem_g, sem_s):
    core = lax.axis_index("c")
    sub = lax.axis_index("s")
    pltpu.sync_copy(dst.at[sub], didx2)
    for p in range(POOL_SLABS // 2):
        h = core * (POOL_SLABS // 2) + p
        pltpu.sync_copy(src.at[h, sub], sidx2)
        pltpu.sync_copy(zinit, acc.at[pl.ds(sub * POOL_RPS, POOL_RPS)])
        plsc.subcore_barrier()

        pltpu.async_copy(table.at[sidx2.at[0]], rows2.at[0], sem_g.at[0])

        def step(k, carry):
            slot = lax.rem(k, 2)
            nslot = lax.rem(k + 1, 2)

            @pl.when(k > 0)
            def _wait_prev_scatter():
                pltpu.make_async_copy(
                    rows2.at[nslot], acc.at[didx2.at[k - 1]], sem_s.at[nslot]
                ).wait()

            @pl.when(k + 1 < POOL_NCHUNK)
            def _fire_next_gather():
                pltpu.async_copy(
                    table.at[sidx2.at[k + 1]], rows2.at[nslot], sem_g.at[nslot]
                )

            pltpu.make_async_copy(
                table.at[sidx2.at[k]], rows2.at[slot], sem_g.at[slot]
            ).wait()
            pltpu.async_copy(
                rows2.at[slot], acc.at[didx2.at[k]], sem_s.at[slot], add=True
            )
            return carry

        lax.fori_loop(0, POOL_NCHUNK, step, 0)
        pltpu.make_async_copy(
            rows2.at[(POOL_NCHUNK - 1) % 2],
            acc.at[didx2.at[POOL_NCHUNK - 1]],
            sem_s.at[(POOL_NCHUNK - 1) % 2],
        ).wait()
        plsc.subcore_barrier()
        pltpu.sync_copy(
            acc.at[pl.ds(sub * POOL_RPS, POOL_RPS)],
            out.at[h, pl.ds(sub * POOL_RPS, POOL_RPS)],
        )
        plsc.subcore_barrier()


BM = 512
NB = (N + BM - 1) // BM
NBG = 4


def _finalize_deg(degfull):
    def body(d_ref, o_ref):
        o_ref[...] = lax.rsqrt(d_ref[0] + d_ref[1])

    return pl.pallas_call(
        body,
        out_shape=jax.ShapeDtypeStruct((NP, 128), jnp.float32),
        grid=(1,),
        in_specs=[pl.BlockSpec((2, NP, 128), lambda i: (0, 0, 0))],
        out_specs=pl.BlockSpec((NP, 128), lambda i: (0, 0)),
    )(degfull)


def _enc_a1(x19, dis, wenc, benc, w1):
    def body(x_ref, dis_ref, we_ref, be_ref, w1_ref, n0_ref, y2_ref):
        n0 = jnp.maximum(
            jnp.dot(x_ref[...], we_ref[...], preferred_element_type=jnp.float32)
            + be_ref[...],
            0.0,
        )
        n0_ref[...] = n0
        d = dis_ref[:, 0:1]
        y = jnp.dot(n0, w1_ref[...], preferred_element_type=jnp.float32) * d
        y2_ref[0] = y[:, :128]
        y2_ref[1] = y[:, 128:]

    return pl.pallas_call(
        body,
        out_shape=(
            jax.ShapeDtypeStruct((N, 512), jnp.float32),
            jax.ShapeDtypeStruct((2, NP, 128), jnp.float32),
        ),
        grid=(NB,),
        in_specs=[
            pl.BlockSpec((BM, 19), lambda i: (i, 0)),
            pl.BlockSpec((BM, 128), lambda i: (i, 0)),
            pl.BlockSpec((19, 512), lambda i: (0, 0)),
            pl.BlockSpec((1, 512), lambda i: (0, 0)),
            pl.BlockSpec((512, 256), lambda i: (0, 0)),
        ],
        out_specs=(
            pl.BlockSpec((BM, 512), lambda i: (i, 0)),
            pl.BlockSpec((2, BM, 128), lambda i: (0, i, 0)),
        ),
    )(x19, dis, wenc, benc, w1)


def _gcn_post(acc2, dis, b, wn=None, want_h=False):
    def body(*refs):
        if wn is not None:
            a_ref, dis_ref, b_ref, wn_ref = refs[:4]
            orefs = refs[4:]
        else:
            a_ref, dis_ref, b_ref = refs[:3]
            orefs = refs[3:]
        d = dis_ref[:, 0:1]
        a = jnp.concatenate([a_ref[0], a_ref[1]], axis=1)
        h = jnp.maximum(a * d + b_ref[...], 0.0)
        i = 0
        if want_h:
            orefs[i][...] = h
            i += 1
        if wn is not None:
            y = jnp.dot(h, wn_ref[...], preferred_element_type=jnp.float32) * d
            orefs[i][0] = y[:, :128]
            orefs[i][1] = y[:, 128:]

    in_specs = [
        pl.BlockSpec((2, BM, 128), lambda i: (0, i, 0)),
        pl.BlockSpec((BM, 128), lambda i: (i, 0)),
        pl.BlockSpec((1, 256), lambda i: (0, 0)),
    ]
    args = [acc2, dis, b]
    if wn is not None:
        in_specs.append(pl.BlockSpec((256, 256), lambda i: (0, 0)))
        args.append(wn)
    out_shape, out_specs = [], []
    if want_h:
        out_shape.append(jax.ShapeDtypeStruct((N, 256), jnp.float32))
        out_specs.append(pl.BlockSpec((BM, 256), lambda i: (i, 0)))
    if wn is not None:
        out_shape.append(jax.ShapeDtypeStruct((2, NP, 128), jnp.float32))
        out_specs.append(pl.BlockSpec((2, BM, 128), lambda i: (0, i, 0)))
    return pl.pallas_call(
        body,
        out_shape=tuple(out_shape),
        grid=(NB,),
        in_specs=in_specs,
        out_specs=tuple(out_specs),
    )(*args)


def _latent(pooled, eps, aggw, aggb, muw, mub, varw, varb, dftw, dftb):
    def body(p_ref, e_ref, aw, ab, mw, mb, vw, vb, dw, db,
             mu_ref, lv_ref, dft_ref):
        cnt = p_ref[10][:, 0:1]
        g = jnp.concatenate(
            [p_ref[h] for h in range(10)], axis=1
        ) / jnp.maximum(cnt, 1.0)
        latent = jnp.dot(g, aw[...], preferred_element_type=jnp.float32) + ab[...]
        mu = jnp.dot(latent, mw[...], preferred_element_type=jnp.float32) + mb[...]
        lv = jnp.dot(latent, vw[...], preferred_element_type=jnp.float32) + vb[...]
        mu_ref[...] = mu
        lv_ref[...] = lv
        z = e_ref[...] * jnp.exp(0.5 * lv) + mu
        dft_ref[...] = jnp.dot(z, dw[...], preferred_element_type=jnp.float32) + db[...]

    return pl.pallas_call(
        body,
        out_shape=(
            jax.ShapeDtypeStruct((G, 256), jnp.float32),
            jax.ShapeDtypeStruct((G, 256), jnp.float32),
            jax.ShapeDtypeStruct((G, 1536), jnp.float32),
        ),
        grid=(NBG,),
        in_specs=[
            pl.BlockSpec((POOL_SLABS, BM, 128), lambda i: (0, i, 0)),
            pl.BlockSpec((BM, 256), lambda i: (i, 0)),
            pl.BlockSpec((1280, 256), lambda i: (0, 0)),
            pl.BlockSpec((1, 256), lambda i: (0, 0)),
            pl.BlockSpec((256, 256), lambda i: (0, 0)),
            pl.BlockSpec((1, 256), lambda i: (0, 0)),
            pl.BlockSpec((256, 256), lambda i: (0, 0)),
            pl.BlockSpec((1, 256), lambda i: (0, 0)),
            pl.BlockSpec((256, 1536), lambda i: (0, 0)),
            pl.BlockSpec((1, 1536), lambda i: (0, 0)),
        ],
        out_specs=(
            pl.BlockSpec((BM, 256), lambda i: (i, 0)),
            pl.BlockSpec((BM, 256), lambda i: (i, 0)),
            pl.BlockSpec((BM, 1536), lambda i: (i, 0)),
        ),
    )(pooled, eps, aggw, aggb, muw, mub, varw, varb, dftw, dftb)


def _dec_a(z6, dis, w):
    def body(z_ref, dis_ref, w_ref, y2_ref):
        d = dis_ref[:, 0:1]
        h = jnp.maximum(z_ref[...], 0.0)
        y = jnp.dot(h, w_ref[...], preferred_element_type=jnp.float32) * d
        y2_ref[0] = y[:, :128]
        y2_ref[1] = y[:, 128:]

    return pl.pallas_call(
        body,
        out_shape=jax.ShapeDtypeStruct((2, NP, 128), jnp.float32),
        grid=(NB,),
        in_specs=[
            pl.BlockSpec((BM, 256), lambda i: (i, 0)),
            pl.BlockSpec((BM, 128), lambda i: (i, 0)),
            pl.BlockSpec((256, 256), lambda i: (0, 0)),
        ],
        out_specs=pl.BlockSpec((2, BM, 128), lambda i: (0, i, 0)),
    )(z6, dis, w)


def _heads(acc2, dis, b3, w0cat, b0cat, bdp, b1p):
    def body(a_ref, dis_ref, b3_ref, w0_ref, b0_ref, bd_ref, b1_ref, o_ref):
        d = dis_ref[:, 0:1]
        a = jnp.concatenate([a_ref[0], a_ref[1]], axis=1)
        h3 = jnp.maximum(a * d + b3_ref[...], 0.0)
        hh = jnp.maximum(
            jnp.dot(h3, w0_ref[...], preferred_element_type=jnp.float32)
            + b0_ref[...],
            0.0,
        )
        o_ref[...] = (
            jnp.dot(hh, bd_ref[...], preferred_element_type=jnp.float32)
            + b1_ref[...]
        )

    return pl.pallas_call(
        body,
        out_shape=jax.ShapeDtypeStruct((N, 128), jnp.float32),
        grid=(NB,),
        in_specs=[
            pl.BlockSpec((2, BM, 128), lambda i: (0, i, 0)),
            pl.BlockSpec((BM, 128), lambda i: (i, 0)),
            pl.BlockSpec((1, 256), lambda i: (0, 0)),
            pl.BlockSpec((256, 1280), lambda i: (0, 0)),
            pl.BlockSpec((1, 1280), lambda i: (0, 0)),
            pl.BlockSpec((1280, 128), lambda i: (0, 0)),
            pl.BlockSpec((1, 128), lambda i: (0, 0)),
        ],
        out_specs=pl.BlockSpec((BM, 128), lambda i: (i, 0)),
    )(acc2, dis, b3, w0cat, b0cat, bdp, b1p)


def _gcn_layer(y2, srcg, dstg, dis, b, wn=None, want_h=False):
    acc = _gcn_sc(y2.reshape(2 * NP, 128), srcg, dstg)
    return _gcn_post(acc.reshape(2, NP, 128), dis, b, wn=wn, want_h=want_h)


def kernel(pos, actor_type, lane_index, direction, params, edge_index, batch):
    p = params
    f32 = jnp.float32
    row, col = edge_index[0], edge_index[1]
    srcg = jnp.concatenate([row, row + NP]).reshape(
        2, NS, GCN_NBLK, GCN_BLK, GCN_CHUNK
    )
    dstg = col.reshape(NS, GCN_NBLK, GCN_BLK, GCN_CHUNK)
    dstd = col.reshape(2, NS, DEG_NCHUNK, DEG_CHUNK)

    wenc = jnp.zeros((19, 512), f32)
    wenc = wenc.at[0:2, 0:128].set(p["pos_W"])
    wenc = wenc.at[2:8, 128:256].set(p["type_W"])
    wenc = wenc.at[8:18, 256:384].set(p["lane_W"])
    wenc = wenc.at[18:19, 384:512].set(p["dir_W"])
    benc = jnp.concatenate(
        [p["pos_b"], p["type_b"], p["lane_b"], p["dir_b"]]
    ).reshape(1, 512)
    x19 = jnp.concatenate([pos, actor_type, lane_index, direction], axis=1)

    w0cat = jnp.concatenate(
        [p["px0_W"], p["py0_W"], p["at0_W"], p["dr0_W"], p["li0_W"]], axis=1
    )
    b0cat = jnp.concatenate(
        [p["px0_b"], p["py0_b"], p["at0_b"], p["dr0_b"], p["li0_b"]]
    ).reshape(1, 1280)
    bdp = jnp.zeros((1280, 128), f32)
    bdp = bdp.at[0:256, 0:1].set(p["px1_W"])
    bdp = bdp.at[256:512, 1:2].set(p["py1_W"])
    bdp = bdp.at[512:768, 2:8].set(p["at1_W"])
    bdp = bdp.at[768:1024, 8:9].set(p["dr1_W"])
    bdp = bdp.at[1024:1280, 9:19].set(p["li1_W"])
    b1p = jnp.zeros((128,), f32)
    b1p = b1p.at[0:1].set(p["px1_b"])
    b1p = b1p.at[1:2].set(p["py1_b"])
    b1p = b1p.at[2:8].set(p["at1_b"])
    b1p = b1p.at[8:9].set(p["dr1_b"])
    b1p = b1p.at[9:19].set(p["li1_b"])
    b1p = b1p.reshape(1, 128)

    eps = jax.random.normal(jax.random.key(42), (G, 256), dtype=f32)

    degfull = _deg_sc(
        dstd,
        jnp.full((ROWS_PER_SUB, 128), 0.5, f32),
        jnp.ones((DEG_CHUNK, 128), f32),
    )
    dis = _finalize_deg(degfull)

    n0, y2 = _enc_a1(x19, dis, wenc, benc, p["e1_W"])
    h1, y2 = _gcn_layer(y2, srcg, dstg, dis, p["e1_b"].reshape(1, 256),
                        wn=p["e2_W"], want_h=True)
    h2, y2 = _gcn_layer(y2, srcg, dstg, dis, p["e2_b"].reshape(1, 256),
                        wn=p["e3_W"], want_h=True)
    (h3,) = _gcn_layer(y2, srcg, dstg, dis, p["e3_b"].reshape(1, 256),
                       wn=None, want_h=True)

    xcat = jnp.concatenate(
        [n0, h1, h2, h3, jnp.ones((N, 256), f32)], axis=1
    )
    ptab = jnp.concatenate(
        [xcat[:, h * 128:(h + 1) * 128] for h in range(POOL_SLABS)], axis=0
    )
    ar = jnp.arange(N, dtype=jnp.int32)
    pad_src = jnp.zeros((POOL_ITEMS - N,), jnp.int32)
    srcp_half = jnp.concatenate([ar, pad_src])
    srcp = jnp.concatenate(
        [srcp_half + h * N for h in range(POOL_SLABS)]
    ).reshape(POOL_SLABS, NS, POOL_NCHUNK, POOL_CHUNK)
    batch_pad = jnp.concatenate(
        [batch, jnp.full((POOL_ITEMS - N,), G, jnp.int32)]
    ).reshape(NS, POOL_NCHUNK, POOL_CHUNK)
    pooled = _pool_sc(ptab, srcp, batch_pad, jnp.zeros((POOL_RPS, 128), f32))

    mu, lv, dft = _latent(
        pooled, eps,
        p["agg_W"], p["agg_b"].reshape(1, 256),
        p["mu_W"], p["mu_b"].reshape(1, 256),
        p["var_W"], p["var_b"].reshape(1, 256),
        p["dft_W"], p["dft_b"].reshape(1, 1536),
    )

    y2 = _dec_a(dft.reshape(N, 256), dis, p["d1_W"])
    (y2,) = _gcn_layer(y2, srcg, dstg, dis, p["d1_b"].reshape(1, 256),
                       wn=p["d2_W"], want_h=False)
    (y2,) = _gcn_layer(y2, srcg, dstg, dis, p["d2_b"].reshape(1, 256),
                       wn=p["d3_W"], want_h=False)
    accd3 = _gcn_sc(y2.reshape(2 * NP, 128), srcg, dstg).reshape(2, NP, 128)
    out = _heads(accd3, dis, p["d3_b"].reshape(1, 256), w0cat, b0cat, bdp, b1p)

    pos_out = out[:, 0:2]
    acttype = out[:, 2:8]
    direc = out[:, 8:9]
    laneidx = out[:, 9:19]
    return (pos_out, acttype, direc, laneidx, lv, mu)


# speedup vs baseline: 8.2586x; 1.0344x over previous
import functools

import jax
import jax.numpy as jnp
from jax import lax
from jax.experimental import pallas as pl
from jax.experimental.pallas import tpu as pltpu
from jax.experimental.pallas import tpu_sc as plsc

N = 12000
E = 192000
G = 2000
NP = 12288

NC, NS = 2, 16

_MESH = plsc.VectorSubcoreMesh(
    core_axis_name="c", subcore_axis_name="s", num_cores=NC, num_subcores=NS
)


ROWS_PER_SUB = NP // NS
EDGES_PER_TILE = E // NS
GCN_CHUNK = 60
GCN_BLK = 25
GCN_NBLK = 8


@functools.partial(
    pl.kernel,
    out_type=jax.ShapeDtypeStruct((2 * NP, 128), jnp.float32),
    mesh=_MESH,
    scratch_types=[
        pltpu.VMEM((GCN_BLK, GCN_CHUNK), jnp.int32),
        pltpu.VMEM((GCN_BLK, GCN_CHUNK), jnp.int32),
        pltpu.VMEM((3, GCN_CHUNK, 128), jnp.float32),
        pltpu.VMEM_SHARED((NP, 128), jnp.float32),
        pltpu.SemaphoreType.DMA((3,)),
        pltpu.SemaphoreType.DMA((3,)),
    ],
)
def _gcn_sc(table, src, dst, out, sidx2, didx2, rows2, acc, sem_g, sem_s):
    core = lax.axis_index("c")
    sub = lax.axis_index("s")
    pltpu.sync_copy(
        table.at[pl.ds(core * NP + sub * ROWS_PER_SUB, ROWS_PER_SUB)],
        acc.at[pl.ds(sub * ROWS_PER_SUB, ROWS_PER_SUB)],
    )
    plsc.subcore_barrier()

    for b in range(GCN_NBLK):
        pltpu.sync_copy(src.at[core, sub, b], sidx2)
        pltpu.sync_copy(dst.at[sub, b], didx2)
        pltpu.async_copy(table.at[sidx2.at[0]], rows2.at[0], sem_g.at[0])
        pltpu.async_copy(table.at[sidx2.at[1]], rows2.at[1], sem_g.at[1])

        def step(k, carry):
            slot = lax.rem(k, 3)
            pltpu.make_async_copy(
                table.at[sidx2.at[k]], rows2.at[slot], sem_g.at[slot]
            ).wait()
            pltpu.async_copy(
                rows2.at[slot], acc.at[didx2.at[k]], sem_s.at[slot], add=True
            )

            @pl.when(k + 2 < GCN_BLK)
            def _fire_next_gather():
                nslot = lax.rem(k + 2, 3)

                @pl.when(k > 0)
                def _wait_prev_scatter():
                    pltpu.make_async_copy(
                        rows2.at[nslot], acc.at[didx2.at[k - 1]], sem_s.at[nslot]
                    ).wait()

                pltpu.async_copy(
                    table.at[sidx2.at[k + 2]], rows2.at[nslot], sem_g.at[nslot]
                )

            return carry

        lax.fori_loop(0, GCN_BLK, step, 0)
        for t in (GCN_BLK - 3, GCN_BLK - 2, GCN_BLK - 1):
            pltpu.make_async_copy(
                rows2.at[t % 3], acc.at[didx2.at[t]], sem_s.at[t % 3]
            ).wait()

    plsc.subcore_barrier()
    pltpu.sync_copy(
        acc.at[pl.ds(sub * ROWS_PER_SUB, ROWS_PER_SUB)],
        out.at[pl.ds(core * NP + sub * ROWS_PER_SUB, ROWS_PER_SUB)],
    )


DEG_CHUNK = 120
DEG_NCHUNK = (E // 2) // NS // DEG_CHUNK


@functools.partial(
    pl.kernel,
    out_type=jax.ShapeDtypeStruct((2, NP, 128), jnp.float32),
    mesh=_MESH,
    scratch_types=[
        pltpu.VMEM((DEG_NCHUNK, DEG_CHUNK), jnp.int32),
        pltpu.VMEM((DEG_CHUNK, 128), jnp.float32),
        pltpu.VMEM_SHARED((NP, 128), jnp.float32),
        pltpu.SemaphoreType.DMA((4,)),
    ],
)
def _deg_sc(dst, half_init, ones_t, out, didx2, ones_v, acc, sem_s):
    core = lax.axis_index("c")
    sub = lax.axis_index("s")
    pltpu.sync_copy(dst.at[core, sub], didx2)
    pltpu.sync_copy(ones_t, ones_v)
    pltpu.sync_copy(
        half_init, acc.at[pl.ds(sub * ROWS_PER_SUB, ROWS_PER_SUB)]
    )
    plsc.subcore_barrier()

    def step(k, carry):
        slot = lax.rem(k, 4)

        @pl.when(k > 3)
        def _wait_prev():
            pltpu.make_async_copy(
                ones_v, acc.at[didx2.at[k - 4]], sem_s.at[slot]
            ).wait()

        pltpu.async_copy(
            ones_v, acc.at[didx2.at[k]], sem_s.at[slot], add=True
        )
        return carry

    lax.fori_loop(0, DEG_NCHUNK, step, 0)
    for t in range(DEG_NCHUNK - 4, DEG_NCHUNK):
        pltpu.make_async_copy(
            ones_v, acc.at[didx2.at[t]], sem_s.at[t % 4]
        ).wait()
    plsc.subcore_barrier()
    pltpu.sync_copy(
        acc.at[pl.ds(sub * ROWS_PER_SUB, ROWS_PER_SUB)],
        out.at[core, pl.ds(sub * ROWS_PER_SUB, ROWS_PER_SUB)],
    )


POOL_ROWS = 2048
POOL_ITEMS = 12288
POOL_PER_TILE = POOL_ITEMS // NS
POOL_CHUNK = 96
POOL_NCHUNK = POOL_PER_TILE // POOL_CHUNK
POOL_RPS = POOL_ROWS // NS
POOL_D = 128
POOL_SLABS = 12


@functools.partial(
    pl.kernel,
    out_type=jax.ShapeDtypeStruct((POOL_SLABS, POOL_ROWS, POOL_D), jnp.float32),
    mesh=_MESH,
    scratch_types=[
        pltpu.VMEM((POOL_NCHUNK, POOL_CHUNK), jnp.int32),
        pltpu.VMEM((POOL_NCHUNK, POOL_CHUNK), jnp.int32),
        pltpu.VMEM((2, POOL_CHUNK, POOL_D), jnp.float32),
        pltpu.VMEM_SHARED((POOL_ROWS, POOL_D), jnp.float32),
        pltpu.SemaphoreType.DMA((2,)),
        pltpu.SemaphoreType.DMA((2,)),
    ],
)
def _pool_sc(table, src, dst, zinit, out, sidx2, didx2, rows2, acc, sem_g, sem_s):
    core = lax.axis_index("c")
    sub = lax.axis_index("s")
    pltpu.sync_copy(dst.at[sub], didx2)
    for p in range(POOL_SLABS // 2):
        h = core * (POOL_SLABS // 2) + p
        pltpu.sync_copy(src.at[h, sub], sidx2)
        pltpu.sync_copy(zinit, acc.at[pl.ds(sub * POOL_RPS, POOL_RPS)])
        plsc.subcore_barrier()

        pltpu.async_copy(table.at[sidx2.at[0]], rows2.at[0], sem_g.at[0])

        def step(k, carry):
            slot = lax.rem(k, 2)
            nslot = lax.rem(k + 1, 2)

            @pl.when(k > 0)
            def _wait_prev_scatter():
                pltpu.make_async_copy(
                    rows2.at[nslot], acc.at[didx2.at[k - 1]], sem_s.at[nslot]
                ).wait()

            @pl.when(k + 1 < POOL_NCHUNK)
            def _fire_next_gather():
                pltpu.async_copy(
                    table.at[sidx2.at[k + 1]], rows2.at[nslot], sem_g.at[nslot]
                )

            pltpu.make_async_copy(
                table.at[sidx2.at[k]], rows2.at[slot], sem_g.at[slot]
            ).wait()
            pltpu.async_copy(
                rows2.at[slot], acc.at[didx2.at[k]], sem_s.at[slot], add=True
            )
            return carry

        lax.fori_loop(0, POOL_NCHUNK, step, 0)
        pltpu.make_async_copy(
            rows2.at[(POOL_NCHUNK - 1) % 2],
            acc.at[didx2.at[POOL_NCHUNK - 1]],
            sem_s.at[(POOL_NCHUNK - 1) % 2],
        ).wait()
        plsc.subcore_barrier()
        pltpu.sync_copy(
            acc.at[pl.ds(sub * POOL_RPS, POOL_RPS)],
            out.at[h, pl.ds(sub * POOL_RPS, POOL_RPS)],
        )
        plsc.subcore_barrier()


BM = 512
NB = (N + BM - 1) // BM
NBG = 4


def _finalize_deg(degfull):
    def body(d_ref, o_ref):
        o_ref[...] = lax.rsqrt(d_ref[0] + d_ref[1])

    return pl.pallas_call(
        body,
        out_shape=jax.ShapeDtypeStruct((NP, 128), jnp.float32),
        grid=(1,),
        in_specs=[pl.BlockSpec((2, NP, 128), lambda i: (0, 0, 0))],
        out_specs=pl.BlockSpec((NP, 128), lambda i: (0, 0)),
    )(degfull)


def _enc_a1(x19, dis, wenc, benc, w1):
    def body(x_ref, dis_ref, we_ref, be_ref, w1_ref, n0_ref, y2_ref):
        n0 = jnp.maximum(
            jnp.dot(x_ref[...], we_ref[...], preferred_element_type=jnp.float32)
            + be_ref[...],
            0.0,
        )
        n0_ref[...] = n0
        d = dis_ref[:, 0:1]
        y = jnp.dot(n0, w1_ref[...], preferred_element_type=jnp.float32) * d
        y2_ref[0] = y[:, :128]
        y2_ref[1] = y[:, 128:]

    return pl.pallas_call(
        body,
        out_shape=(
            jax.ShapeDtypeStruct((N, 512), jnp.float32),
            jax.ShapeDtypeStruct((2, NP, 128), jnp.float32),
        ),
        grid=(NB,),
        in_specs=[
            pl.BlockSpec((BM, 19), lambda i: (i, 0)),
            pl.BlockSpec((BM, 128), lambda i: (i, 0)),
            pl.BlockSpec((19, 512), lambda i: (0, 0)),
            pl.BlockSpec((1, 512), lambda i: (0, 0)),
            pl.BlockSpec((512, 256), lambda i: (0, 0)),
        ],
        out_specs=(
            pl.BlockSpec((BM, 512), lambda i: (i, 0)),
            pl.BlockSpec((2, BM, 128), lambda i: (0, i, 0)),
        ),
    )(x19, dis, wenc, benc, w1)


def _gcn_post(acc2, dis, b, wn=None, want_h=False):
    def body(*refs):
        if wn is not None:
            a_ref, dis_ref, b_ref, wn_ref = refs[:4]
            orefs = refs[4:]
        else:
            a_ref, dis_ref, b_ref = refs[:3]
            orefs = refs[3:]
        d = dis_ref[:, 0:1]
        a = jnp.concatenate([a_ref[0], a_ref[1]], axis=1)
        h = jnp.maximum(a * d + b_ref[...], 0.0)
        i = 0
        if want_h:
            orefs[i][...] = h
            i += 1
        if wn is not None:
            y = jnp.dot(h, wn_ref[...], preferred_element_type=jnp.float32) * d
            orefs[i][0] = y[:, :128]
            orefs[i][1] = y[:, 128:]

    in_specs = [
        pl.BlockSpec((2, BM, 128), lambda i: (0, i, 0)),
        pl.BlockSpec((BM, 128), lambda i: (i, 0)),
        pl.BlockSpec((1, 256), lambda i: (0, 0)),
    ]
    args = [acc2, dis, b]
    if wn is not None:
        in_specs.append(pl.BlockSpec((256, 256), lambda i: (0, 0)))
        args.append(wn)
    out_shape, out_specs = [], []
    if want_h:
        out_shape.append(jax.ShapeDtypeStruct((N, 256), jnp.float32))
        out_specs.append(pl.BlockSpec((BM, 256), lambda i: (i, 0)))
    if wn is not None:
        out_shape.append(jax.ShapeDtypeStruct((2, NP, 128), jnp.float32))
        out_specs.append(pl.BlockSpec((2, BM, 128), lambda i: (0, i, 0)))
    return pl.pallas_call(
        body,
        out_shape=tuple(out_shape),
        grid=(NB,),
        in_specs=in_specs,
        out_specs=tuple(out_specs),
    )(*args)


def _latent(pooled, eps, aggw, aggb, muw, mub, varw, varb, dftw, dftb):
    def body(p_ref, e_ref, aw, ab, mw, mb, vw, vb, dw, db,
             mu_ref, lv_ref, dft_ref):
        cnt = p_ref[10][:, 0:1]
        g = jnp.concatenate(
            [p_ref[h] for h in range(10)], axis=1
        ) / jnp.maximum(cnt, 1.0)
        latent = jnp.dot(g, aw[...], preferred_element_type=jnp.float32) + ab[...]
        mu = jnp.dot(latent, mw[...], preferred_element_type=jnp.float32) + mb[...]
        lv = jnp.dot(latent, vw[...], preferred_element_type=jnp.float32) + vb[...]
        mu_ref[...] = mu
        lv_ref[...] = lv
        z = e_ref[...] * jnp.exp(0.5 * lv) + mu
        dft_ref[...] = jnp.dot(z, dw[...], preferred_element_type=jnp.float32) + db[...]

    return pl.pallas_call(
        body,
        out_shape=(
            jax.ShapeDtypeStruct((G, 256), jnp.float32),
            jax.ShapeDtypeStruct((G, 256), jnp.float32),
            jax.ShapeDtypeStruct((G, 1536), jnp.float32),
        ),
        grid=(NBG,),
        in_specs=[
            pl.BlockSpec((POOL_SLABS, BM, POOL_D), lambda i: (0, i, 0)),
            pl.BlockSpec((BM, 256), lambda i: (i, 0)),
            pl.BlockSpec((1280, 256), lambda i: (0, 0)),
            pl.BlockSpec((1, 256), lambda i: (0, 0)),
            pl.BlockSpec((256, 256), lambda i: (0, 0)),
            pl.BlockSpec((1, 256), lambda i: (0, 0)),
            pl.BlockSpec((256, 256), lambda i: (0, 0)),
            pl.BlockSpec((1, 256), lambda i: (0, 0)),
            pl.BlockSpec((256, 1536), lambda i: (0, 0)),
            pl.BlockSpec((1, 1536), lambda i: (0, 0)),
        ],
        out_specs=(
            pl.BlockSpec((BM, 256), lambda i: (i, 0)),
            pl.BlockSpec((BM, 256), lambda i: (i, 0)),
            pl.BlockSpec((BM, 1536), lambda i: (i, 0)),
        ),
    )(pooled, eps, aggw, aggb, muw, mub, varw, varb, dftw, dftb)


def _dec_a(z6, dis, w):
    def body(z_ref, dis_ref, w_ref, y2_ref):
        d = dis_ref[:, 0:1]
        h = jnp.maximum(z_ref[...], 0.0)
        y = jnp.dot(h, w_ref[...], preferred_element_type=jnp.float32) * d
        y2_ref[0] = y[:, :128]
        y2_ref[1] = y[:, 128:]

    return pl.pallas_call(
        body,
        out_shape=jax.ShapeDtypeStruct((2, NP, 128), jnp.float32),
        grid=(NB,),
        in_specs=[
            pl.BlockSpec((BM, 256), lambda i: (i, 0)),
            pl.BlockSpec((BM, 128), lambda i: (i, 0)),
            pl.BlockSpec((256, 256), lambda i: (0, 0)),
        ],
        out_specs=pl.BlockSpec((2, BM, 128), lambda i: (0, i, 0)),
    )(z6, dis, w)


def _heads(acc2, dis, b3, w0cat, b0cat, bdp, b1p):
    def body(a_ref, dis_ref, b3_ref, w0_ref, b0_ref, bd_ref, b1_ref, o_ref):
        d = dis_ref[:, 0:1]
        a = jnp.concatenate([a_ref[0], a_ref[1]], axis=1)
        h3 = jnp.maximum(a * d + b3_ref[...], 0.0)
        hh = jnp.maximum(
            jnp.dot(h3, w0_ref[...], preferred_element_type=jnp.float32)
            + b0_ref[...],
            0.0,
        )
        o_ref[...] = (
            jnp.dot(hh, bd_ref[...], preferred_element_type=jnp.float32)
            + b1_ref[...]
        )

    return pl.pallas_call(
        body,
        out_shape=jax.ShapeDtypeStruct((N, 128), jnp.float32),
        grid=(NB,),
        in_specs=[
            pl.BlockSpec((2, BM, 128), lambda i: (0, i, 0)),
            pl.BlockSpec((BM, 128), lambda i: (i, 0)),
            pl.BlockSpec((1, 256), lambda i: (0, 0)),
            pl.BlockSpec((256, 1280), lambda i: (0, 0)),
            pl.BlockSpec((1, 1280), lambda i: (0, 0)),
            pl.BlockSpec((1280, 128), lambda i: (0, 0)),
            pl.BlockSpec((1, 128), lambda i: (0, 0)),
        ],
        out_specs=pl.BlockSpec((BM, 128), lambda i: (i, 0)),
    )(acc2, dis, b3, w0cat, b0cat, bdp, b1p)


def _gcn_layer(y2, srcg, dstg, dis, b, wn=None, want_h=False):
    acc = _gcn_sc(y2.reshape(2 * NP, 128), srcg, dstg)
    return _gcn_post(acc.reshape(2, NP, 128), dis, b, wn=wn, want_h=want_h)


def kernel(pos, actor_type, lane_index, direction, params, edge_index, batch):
    p = params
    f32 = jnp.float32
    row, col = edge_index[0], edge_index[1]
    srcg = jnp.concatenate([row, row + NP]).reshape(
        2, NS, GCN_NBLK, GCN_BLK, GCN_CHUNK
    )
    dstg = col.reshape(NS, GCN_NBLK, GCN_BLK, GCN_CHUNK)
    dstd = col.reshape(2, NS, DEG_NCHUNK, DEG_CHUNK)

    wenc = jnp.zeros((19, 512), f32)
    wenc = wenc.at[0:2, 0:128].set(p["pos_W"])
    wenc = wenc.at[2:8, 128:256].set(p["type_W"])
    wenc = wenc.at[8:18, 256:384].set(p["lane_W"])
    wenc = wenc.at[18:19, 384:512].set(p["dir_W"])
    benc = jnp.concatenate(
        [p["pos_b"], p["type_b"], p["lane_b"], p["dir_b"]]
    ).reshape(1, 512)
    x19 = jnp.concatenate([pos, actor_type, lane_index, direction], axis=1)

    w0cat = jnp.concatenate(
        [p["px0_W"], p["py0_W"], p["at0_W"], p["dr0_W"], p["li0_W"]], axis=1
    )
    b0cat = jnp.concatenate(
        [p["px0_b"], p["py0_b"], p["at0_b"], p["dr0_b"], p["li0_b"]]
    ).reshape(1, 1280)
    bdp = jnp.zeros((1280, 128), f32)
    bdp = bdp.at[0:256, 0:1].set(p["px1_W"])
    bdp = bdp.at[256:512, 1:2].set(p["py1_W"])
    bdp = bdp.at[512:768, 2:8].set(p["at1_W"])
    bdp = bdp.at[768:1024, 8:9].set(p["dr1_W"])
    bdp = bdp.at[1024:1280, 9:19].set(p["li1_W"])
    b1p = jnp.zeros((128,), f32)
    b1p = b1p.at[0:1].set(p["px1_b"])
    b1p = b1p.at[1:2].set(p["py1_b"])
    b1p = b1p.at[2:8].set(p["at1_b"])
    b1p = b1p.at[8:9].set(p["dr1_b"])
    b1p = b1p.at[9:19].set(p["li1_b"])
    b1p = b1p.reshape(1, 128)

    eps = jax.random.normal(jax.random.key(42), (G, 256), dtype=f32)

    degfull = _deg_sc(
        dstd,
        jnp.full((ROWS_PER_SUB, 128), 0.5, f32),
        jnp.ones((DEG_CHUNK, 128), f32),
    )
    dis = _finalize_deg(degfull)

    n0, y2 = _enc_a1(x19, dis, wenc, benc, p["e1_W"])
    h1, y2 = _gcn_layer(y2, srcg, dstg, dis, p["e1_b"].reshape(1, 256),
                        wn=p["e2_W"], want_h=True)
    h2, y2 = _gcn_layer(y2, srcg, dstg, dis, p["e2_b"].reshape(1, 256),
                        wn=p["e3_W"], want_h=True)
    (h3,) = _gcn_layer(y2, srcg, dstg, dis, p["e3_b"].reshape(1, 256),
                       wn=None, want_h=True)

    xcat = jnp.concatenate(
        [n0, h1, h2, h3, jnp.ones((N, 256), f32)], axis=1
    )
    ptab = jnp.concatenate(
        [xcat[:, h * POOL_D:(h + 1) * POOL_D] for h in range(POOL_SLABS)],
        axis=0,
    )
    ar = jnp.arange(N, dtype=jnp.int32)
    pad_src = jnp.zeros((POOL_ITEMS - N,), jnp.int32)
    srcp_half = jnp.concatenate([ar, pad_src])
    srcp = jnp.concatenate(
        [srcp_half + h * N for h in range(POOL_SLABS)]
    ).reshape(POOL_SLABS, NS, POOL_NCHUNK, POOL_CHUNK)
    batch_pad = jnp.concatenate(
        [batch, jnp.full((POOL_ITEMS - N,), G, jnp.int32)]
    ).reshape(NS, POOL_NCHUNK, POOL_CHUNK)
    pooled = _pool_sc(ptab, srcp, batch_pad, jnp.zeros((POOL_RPS, POOL_D), f32))

    mu, lv, dft = _latent(
        pooled, eps,
        p["agg_W"], p["agg_b"].reshape(1, 256),
        p["mu_W"], p["mu_b"].reshape(1, 256),
        p["var_W"], p["var_b"].reshape(1, 256),
        p["dft_W"], p["dft_b"].reshape(1, 1536),
    )

    y2 = _dec_a(dft.reshape(N, 256), dis, p["d1_W"])
    (y2,) = _gcn_layer(y2, srcg, dstg, dis, p["d1_b"].reshape(1, 256),
                       wn=p["d2_W"], want_h=False)
    (y2,) = _gcn_layer(y2, srcg, dstg, dis, p["d2_b"].reshape(1, 256),
                       wn=p["d3_W"], want_h=False)
    accd3 = _gcn_sc(y2.reshape(2 * NP, 128), srcg, dstg).reshape(2, NP, 128)
    out = _heads(accd3, dis, p["d3_b"].reshape(1, 256), w0cat, b0cat, bdp, b1p)

    pos_out = out[:, 0:2]
    acttype = out[:, 2:8]
    direc = out[:, 8:9]
    laneidx = out[:, 9:19]
    return (pos_out, acttype, direc, laneidx, lv, mu)


# speedup vs baseline: 8.2723x; 1.0017x over previous
import functools

import jax
import jax.numpy as jnp
from jax import lax
from jax.experimental import pallas as pl
from jax.experimental.pallas import tpu as pltpu
from jax.experimental.pallas import tpu_sc as plsc

N = 12000
E = 192000
G = 2000
NP = 12288

NC, NS = 2, 16

_MESH = plsc.VectorSubcoreMesh(
    core_axis_name="c", subcore_axis_name="s", num_cores=NC, num_subcores=NS
)


ROWS_PER_SUB = NP // NS
EDGES_PER_TILE = E // NS
GCN_CHUNK = 60
GCN_BLK = 25
GCN_NBLK = 8


@functools.partial(
    pl.kernel,
    out_type=jax.ShapeDtypeStruct((2 * NP, 128), jnp.float32),
    mesh=_MESH,
    scratch_types=[
        pltpu.VMEM((GCN_BLK, GCN_CHUNK), jnp.int32),
        pltpu.VMEM((GCN_BLK, GCN_CHUNK), jnp.int32),
        pltpu.VMEM((3, GCN_CHUNK, 128), jnp.float32),
        pltpu.VMEM_SHARED((NP, 128), jnp.float32),
        pltpu.SemaphoreType.DMA((3,)),
        pltpu.SemaphoreType.DMA((3,)),
    ],
)
def _gcn_sc(table, src, dst, out, sidx2, didx2, rows2, acc, sem_g, sem_s):
    core = lax.axis_index("c")
    sub = lax.axis_index("s")
    pltpu.sync_copy(
        table.at[pl.ds(core * NP + sub * ROWS_PER_SUB, ROWS_PER_SUB)],
        acc.at[pl.ds(sub * ROWS_PER_SUB, ROWS_PER_SUB)],
    )
    plsc.subcore_barrier()

    def block(b, carry0):
        pltpu.sync_copy(src.at[core, sub, b], sidx2)
        pltpu.sync_copy(dst.at[sub, b], didx2)
        pltpu.async_copy(table.at[sidx2.at[0]], rows2.at[0], sem_g.at[0])
        pltpu.async_copy(table.at[sidx2.at[1]], rows2.at[1], sem_g.at[1])

        def step(k, carry):
            slot = lax.rem(k, 3)
            pltpu.make_async_copy(
                table.at[sidx2.at[k]], rows2.at[slot], sem_g.at[slot]
            ).wait()
            pltpu.async_copy(
                rows2.at[slot], acc.at[didx2.at[k]], sem_s.at[slot], add=True
            )

            @pl.when(k + 2 < GCN_BLK)
            def _fire_next_gather():
                nslot = lax.rem(k + 2, 3)

                @pl.when(k > 0)
                def _wait_prev_scatter():
                    pltpu.make_async_copy(
                        rows2.at[nslot], acc.at[didx2.at[k - 1]], sem_s.at[nslot]
                    ).wait()

                pltpu.async_copy(
                    table.at[sidx2.at[k + 2]], rows2.at[nslot], sem_g.at[nslot]
                )

            return carry

        lax.fori_loop(0, GCN_BLK, step, 0)
        for t in (GCN_BLK - 3, GCN_BLK - 2, GCN_BLK - 1):
            pltpu.make_async_copy(
                rows2.at[t % 3], acc.at[didx2.at[t]], sem_s.at[t % 3]
            ).wait()
        return carry0

    lax.fori_loop(0, GCN_NBLK, block, 0)
    plsc.subcore_barrier()
    pltpu.sync_copy(
        acc.at[pl.ds(sub * ROWS_PER_SUB, ROWS_PER_SUB)],
        out.at[pl.ds(core * NP + sub * ROWS_PER_SUB, ROWS_PER_SUB)],
    )


DEG_CHUNK = 120
DEG_NCHUNK = (E // 2) // NS // DEG_CHUNK


@functools.partial(
    pl.kernel,
    out_type=jax.ShapeDtypeStruct((2, NP, 128), jnp.float32),
    mesh=_MESH,
    scratch_types=[
        pltpu.VMEM((DEG_NCHUNK, DEG_CHUNK), jnp.int32),
        pltpu.VMEM((DEG_CHUNK, 128), jnp.float32),
        pltpu.VMEM_SHARED((NP, 128), jnp.float32),
        pltpu.SemaphoreType.DMA((4,)),
    ],
)
def _deg_sc(dst, half_init, ones_t, out, didx2, ones_v, acc, sem_s):
    core = lax.axis_index("c")
    sub = lax.axis_index("s")
    pltpu.sync_copy(dst.at[core, sub], didx2)
    pltpu.sync_copy(ones_t, ones_v)
    pltpu.sync_copy(
        half_init, acc.at[pl.ds(sub * ROWS_PER_SUB, ROWS_PER_SUB)]
    )
    plsc.subcore_barrier()

    def step(k, carry):
        slot = lax.rem(k, 4)

        @pl.when(k > 3)
        def _wait_prev():
            pltpu.make_async_copy(
                ones_v, acc.at[didx2.at[k - 4]], sem_s.at[slot]
            ).wait()

        pltpu.async_copy(
            ones_v, acc.at[didx2.at[k]], sem_s.at[slot], add=True
        )
        return carry

    lax.fori_loop(0, DEG_NCHUNK, step, 0)
    for t in range(DEG_NCHUNK - 4, DEG_NCHUNK):
        pltpu.make_async_copy(
            ones_v, acc.at[didx2.at[t]], sem_s.at[t % 4]
        ).wait()
    plsc.subcore_barrier()
    pltpu.sync_copy(
        acc.at[pl.ds(sub * ROWS_PER_SUB, ROWS_PER_SUB)],
        out.at[core, pl.ds(sub * ROWS_PER_SUB, ROWS_PER_SUB)],
    )


POOL_ROWS = 2048
POOL_ITEMS = 12288
POOL_PER_TILE = POOL_ITEMS // NS
POOL_CHUNK = 96
POOL_NCHUNK = POOL_PER_TILE // POOL_CHUNK
POOL_RPS = POOL_ROWS // NS
POOL_D = 128
POOL_SLABS = 12


@functools.partial(
    pl.kernel,
    out_type=jax.ShapeDtypeStruct((POOL_SLABS, POOL_ROWS, POOL_D), jnp.float32),
    mesh=_MESH,
    scratch_types=[
        pltpu.VMEM((POOL_NCHUNK, POOL_CHUNK), jnp.int32),
        pltpu.VMEM((POOL_NCHUNK, POOL_CHUNK), jnp.int32),
        pltpu.VMEM((2, POOL_CHUNK, POOL_D), jnp.float32),
        pltpu.VMEM_SHARED((POOL_ROWS, POOL_D), jnp.float32),
        pltpu.SemaphoreType.DMA((2,)),
        pltpu.SemaphoreType.DMA((2,)),
    ],
)
def _pool_sc(table, src, dst, zinit, out, sidx2, didx2, rows2, acc, sem_g, sem_s):
    core = lax.axis_index("c")
    sub = lax.axis_index("s")
    pltpu.sync_copy(dst.at[sub], didx2)

    def ppass(p, carry0):
        h = core * (POOL_SLABS // 2) + p
        pltpu.sync_copy(src.at[h, sub], sidx2)
        pltpu.sync_copy(zinit, acc.at[pl.ds(sub * POOL_RPS, POOL_RPS)])
        plsc.subcore_barrier()

        pltpu.async_copy(table.at[sidx2.at[0]], rows2.at[0], sem_g.at[0])

        def step(k, carry):
            slot = lax.rem(k, 2)
            nslot = lax.rem(k + 1, 2)

            @pl.when(k > 0)
            def _wait_prev_scatter():
                pltpu.make_async_copy(
                    rows2.at[nslot], acc.at[didx2.at[k - 1]], sem_s.at[nslot]
                ).wait()

            @pl.when(k + 1 < POOL_NCHUNK)
            def _fire_next_gather():
                pltpu.async_copy(
                    table.at[sidx2.at[k + 1]], rows2.at[nslot], sem_g.at[nslot]
                )

            pltpu.make_async_copy(
                table.at[sidx2.at[k]], rows2.at[slot], sem_g.at[slot]
            ).wait()
            pltpu.async_copy(
                rows2.at[slot], acc.at[didx2.at[k]], sem_s.at[slot], add=True
            )
            return carry

        lax.fori_loop(0, POOL_NCHUNK, step, 0)
        pltpu.make_async_copy(
            rows2.at[(POOL_NCHUNK - 1) % 2],
            acc.at[didx2.at[POOL_NCHUNK - 1]],
            sem_s.at[(POOL_NCHUNK - 1) % 2],
        ).wait()
        plsc.subcore_barrier()
        pltpu.sync_copy(
            acc.at[pl.ds(sub * POOL_RPS, POOL_RPS)],
            out.at[h, pl.ds(sub * POOL_RPS, POOL_RPS)],
        )
        plsc.subcore_barrier()
        return carry0

    lax.fori_loop(0, POOL_SLABS // 2, ppass, 0)


BM = 512
NB = (N + BM - 1) // BM
NBG = 4


def _finalize_deg(degfull):
    def body(d_ref, o_ref):
        o_ref[...] = lax.rsqrt(d_ref[0] + d_ref[1])

    return pl.pallas_call(
        body,
        out_shape=jax.ShapeDtypeStruct((NP, 128), jnp.float32),
        grid=(1,),
        in_specs=[pl.BlockSpec((2, NP, 128), lambda i: (0, 0, 0))],
        out_specs=pl.BlockSpec((NP, 128), lambda i: (0, 0)),
    )(degfull)


def _enc_a1(x19, dis, wenc, benc, w1):
    def body(x_ref, dis_ref, we_ref, be_ref, w1_ref, n0_ref, y2_ref):
        n0 = jnp.maximum(
            jnp.dot(x_ref[...], we_ref[...], preferred_element_type=jnp.float32)
            + be_ref[...],
            0.0,
        )
        n0_ref[...] = n0
        d = dis_ref[:, 0:1]
        y = jnp.dot(n0, w1_ref[...], preferred_element_type=jnp.float32) * d
        y2_ref[0] = y[:, :128]
        y2_ref[1] = y[:, 128:]

    return pl.pallas_call(
        body,
        out_shape=(
            jax.ShapeDtypeStruct((N, 512), jnp.float32),
            jax.ShapeDtypeStruct((2, NP, 128), jnp.float32),
        ),
        grid=(NB,),
        in_specs=[
            pl.BlockSpec((BM, 19), lambda i: (i, 0)),
            pl.BlockSpec((BM, 128), lambda i: (i, 0)),
            pl.BlockSpec((19, 512), lambda i: (0, 0)),
            pl.BlockSpec((1, 512), lambda i: (0, 0)),
            pl.BlockSpec((512, 256), lambda i: (0, 0)),
        ],
        out_specs=(
            pl.BlockSpec((BM, 512), lambda i: (i, 0)),
            pl.BlockSpec((2, BM, 128), lambda i: (0, i, 0)),
        ),
    )(x19, dis, wenc, benc, w1)


def _gcn_post(acc2, dis, b, wn=None, want_h=False):
    def body(*refs):
        if wn is not None:
            a_ref, dis_ref, b_ref, wn_ref = refs[:4]
            orefs = refs[4:]
        else:
            a_ref, dis_ref, b_ref = refs[:3]
            orefs = refs[3:]
        d = dis_ref[:, 0:1]
        a = jnp.concatenate([a_ref[0], a_ref[1]], axis=1)
        h = jnp.maximum(a * d + b_ref[...], 0.0)
        i = 0
        if want_h:
            orefs[i][...] = h
            i += 1
        if wn is not None:
            y = jnp.dot(h, wn_ref[...], preferred_element_type=jnp.float32) * d
            orefs[i][0] = y[:, :128]
            orefs[i][1] = y[:, 128:]

    in_specs = [
        pl.BlockSpec((2, BM, 128), lambda i: (0, i, 0)),
        pl.BlockSpec((BM, 128), lambda i: (i, 0)),
        pl.BlockSpec((1, 256), lambda i: (0, 0)),
    ]
    args = [acc2, dis, b]
    if wn is not None:
        in_specs.append(pl.BlockSpec((256, 256), lambda i: (0, 0)))
        args.append(wn)
    out_shape, out_specs = [], []
    if want_h:
        out_shape.append(jax.ShapeDtypeStruct((N, 256), jnp.float32))
        out_specs.append(pl.BlockSpec((BM, 256), lambda i: (i, 0)))
    if wn is not None:
        out_shape.append(jax.ShapeDtypeStruct((2, NP, 128), jnp.float32))
        out_specs.append(pl.BlockSpec((2, BM, 128), lambda i: (0, i, 0)))
    return pl.pallas_call(
        body,
        out_shape=tuple(out_shape),
        grid=(NB,),
        in_specs=in_specs,
        out_specs=tuple(out_specs),
    )(*args)


def _latent(pooled, eps, aggw, aggb, muw, mub, varw, varb, dftw, dftb):
    def body(p_ref, e_ref, aw, ab, mw, mb, vw, vb, dw, db,
             mu_ref, lv_ref, dft_ref):
        cnt = p_ref[10][:, 0:1]
        g = jnp.concatenate(
            [p_ref[h] for h in range(10)], axis=1
        ) / jnp.maximum(cnt, 1.0)
        latent = jnp.dot(g, aw[...], preferred_element_type=jnp.float32) + ab[...]
        mu = jnp.dot(latent, mw[...], preferred_element_type=jnp.float32) + mb[...]
        lv = jnp.dot(latent, vw[...], preferred_element_type=jnp.float32) + vb[...]
        mu_ref[...] = mu
        lv_ref[...] = lv
        z = e_ref[...] * jnp.exp(0.5 * lv) + mu
        dft_ref[...] = jnp.dot(z, dw[...], preferred_element_type=jnp.float32) + db[...]

    return pl.pallas_call(
        body,
        out_shape=(
            jax.ShapeDtypeStruct((G, 256), jnp.float32),
            jax.ShapeDtypeStruct((G, 256), jnp.float32),
            jax.ShapeDtypeStruct((G, 1536), jnp.float32),
        ),
        grid=(NBG,),
        in_specs=[
            pl.BlockSpec((POOL_SLABS, BM, POOL_D), lambda i: (0, i, 0)),
            pl.BlockSpec((BM, 256), lambda i: (i, 0)),
            pl.BlockSpec((1280, 256), lambda i: (0, 0)),
            pl.BlockSpec((1, 256), lambda i: (0, 0)),
            pl.BlockSpec((256, 256), lambda i: (0, 0)),
            pl.BlockSpec((1, 256), lambda i: (0, 0)),
            pl.BlockSpec((256, 256), lambda i: (0, 0)),
            pl.BlockSpec((1, 256), lambda i: (0, 0)),
            pl.BlockSpec((256, 1536), lambda i: (0, 0)),
            pl.BlockSpec((1, 1536), lambda i: (0, 0)),
        ],
        out_specs=(
            pl.BlockSpec((BM, 256), lambda i: (i, 0)),
            pl.BlockSpec((BM, 256), lambda i: (i, 0)),
            pl.BlockSpec((BM, 1536), lambda i: (i, 0)),
        ),
    )(pooled, eps, aggw, aggb, muw, mub, varw, varb, dftw, dftb)


def _dec_a(z6, dis, w):
    def body(z_ref, dis_ref, w_ref, y2_ref):
        d = dis_ref[:, 0:1]
        h = jnp.maximum(z_ref[...], 0.0)
        y = jnp.dot(h, w_ref[...], preferred_element_type=jnp.float32) * d
        y2_ref[0] = y[:, :128]
        y2_ref[1] = y[:, 128:]

    return pl.pallas_call(
        body,
        out_shape=jax.ShapeDtypeStruct((2, NP, 128), jnp.float32),
        grid=(NB,),
        in_specs=[
            pl.BlockSpec((BM, 256), lambda i: (i, 0)),
            pl.BlockSpec((BM, 128), lambda i: (i, 0)),
            pl.BlockSpec((256, 256), lambda i: (0, 0)),
        ],
        out_specs=pl.BlockSpec((2, BM, 128), lambda i: (0, i, 0)),
    )(z6, dis, w)


def _heads(acc2, dis, b3, w0cat, b0cat, bdp, b1p):
    def body(a_ref, dis_ref, b3_ref, w0_ref, b0_ref, bd_ref, b1_ref, o_ref):
        d = dis_ref[:, 0:1]
        a = jnp.concatenate([a_ref[0], a_ref[1]], axis=1)
        h3 = jnp.maximum(a * d + b3_ref[...], 0.0)
        hh = jnp.maximum(
            jnp.dot(h3, w0_ref[...], preferred_element_type=jnp.float32)
            + b0_ref[...],
            0.0,
        )
        o_ref[...] = (
            jnp.dot(hh, bd_ref[...], preferred_element_type=jnp.float32)
            + b1_ref[...]
        )

    return pl.pallas_call(
        body,
        out_shape=jax.ShapeDtypeStruct((N, 128), jnp.float32),
        grid=(NB,),
        in_specs=[
            pl.BlockSpec((2, BM, 128), lambda i: (0, i, 0)),
            pl.BlockSpec((BM, 128), lambda i: (i, 0)),
            pl.BlockSpec((1, 256), lambda i: (0, 0)),
            pl.BlockSpec((256, 1280), lambda i: (0, 0)),
            pl.BlockSpec((1, 1280), lambda i: (0, 0)),
            pl.BlockSpec((1280, 128), lambda i: (0, 0)),
            pl.BlockSpec((1, 128), lambda i: (0, 0)),
        ],
        out_specs=pl.BlockSpec((BM, 128), lambda i: (i, 0)),
    )(acc2, dis, b3, w0cat, b0cat, bdp, b1p)


def _gcn_layer(y2, srcg, dstg, dis, b, wn=None, want_h=False):
    acc = _gcn_sc(y2.reshape(2 * NP, 128), srcg, dstg)
    return _gcn_post(acc.reshape(2, NP, 128), dis, b, wn=wn, want_h=want_h)


def kernel(pos, actor_type, lane_index, direction, params, edge_index, batch):
    p = params
    f32 = jnp.float32
    row, col = edge_index[0], edge_index[1]
    srcg = jnp.concatenate([row, row + NP]).reshape(
        2, NS, GCN_NBLK, GCN_BLK, GCN_CHUNK
    )
    dstg = col.reshape(NS, GCN_NBLK, GCN_BLK, GCN_CHUNK)
    dstd = col.reshape(2, NS, DEG_NCHUNK, DEG_CHUNK)

    wenc = jnp.zeros((19, 512), f32)
    wenc = wenc.at[0:2, 0:128].set(p["pos_W"])
    wenc = wenc.at[2:8, 128:256].set(p["type_W"])
    wenc = wenc.at[8:18, 256:384].set(p["lane_W"])
    wenc = wenc.at[18:19, 384:512].set(p["dir_W"])
    benc = jnp.concatenate(
        [p["pos_b"], p["type_b"], p["lane_b"], p["dir_b"]]
    ).reshape(1, 512)
    x19 = jnp.concatenate([pos, actor_type, lane_index, direction], axis=1)

    w0cat = jnp.concatenate(
        [p["px0_W"], p["py0_W"], p["at0_W"], p["dr0_W"], p["li0_W"]], axis=1
    )
    b0cat = jnp.concatenate(
        [p["px0_b"], p["py0_b"], p["at0_b"], p["dr0_b"], p["li0_b"]]
    ).reshape(1, 1280)
    bdp = jnp.zeros((1280, 128), f32)
    bdp = bdp.at[0:256, 0:1].set(p["px1_W"])
    bdp = bdp.at[256:512, 1:2].set(p["py1_W"])
    bdp = bdp.at[512:768, 2:8].set(p["at1_W"])
    bdp = bdp.at[768:1024, 8:9].set(p["dr1_W"])
    bdp = bdp.at[1024:1280, 9:19].set(p["li1_W"])
    b1p = jnp.zeros((128,), f32)
    b1p = b1p.at[0:1].set(p["px1_b"])
    b1p = b1p.at[1:2].set(p["py1_b"])
    b1p = b1p.at[2:8].set(p["at1_b"])
    b1p = b1p.at[8:9].set(p["dr1_b"])
    b1p = b1p.at[9:19].set(p["li1_b"])
    b1p = b1p.reshape(1, 128)

    eps = jax.random.normal(jax.random.key(42), (G, 256), dtype=f32)

    degfull = _deg_sc(
        dstd,
        jnp.full((ROWS_PER_SUB, 128), 0.5, f32),
        jnp.ones((DEG_CHUNK, 128), f32),
    )
    dis = _finalize_deg(degfull)

    n0, y2 = _enc_a1(x19, dis, wenc, benc, p["e1_W"])
    h1, y2 = _gcn_layer(y2, srcg, dstg, dis, p["e1_b"].reshape(1, 256),
                        wn=p["e2_W"], want_h=True)
    h2, y2 = _gcn_layer(y2, srcg, dstg, dis, p["e2_b"].reshape(1, 256),
                        wn=p["e3_W"], want_h=True)
    (h3,) = _gcn_layer(y2, srcg, dstg, dis, p["e3_b"].reshape(1, 256),
                       wn=None, want_h=True)

    xcat = jnp.concatenate(
        [n0, h1, h2, h3, jnp.ones((N, 256), f32)], axis=1
    )
    ptab = jnp.concatenate(
        [xcat[:, h * POOL_D:(h + 1) * POOL_D] for h in range(POOL_SLABS)],
        axis=0,
    )
    ar = jnp.arange(N, dtype=jnp.int32)
    pad_src = jnp.zeros((POOL_ITEMS - N,), jnp.int32)
    srcp_half = jnp.concatenate([ar, pad_src])
    srcp = jnp.concatenate(
        [srcp_half + h * N for h in range(POOL_SLABS)]
    ).reshape(POOL_SLABS, NS, POOL_NCHUNK, POOL_CHUNK)
    batch_pad = jnp.concatenate(
        [batch, jnp.full((POOL_ITEMS - N,), G, jnp.int32)]
    ).reshape(NS, POOL_NCHUNK, POOL_CHUNK)
    pooled = _pool_sc(ptab, srcp, batch_pad, jnp.zeros((POOL_RPS, POOL_D), f32))

    mu, lv, dft = _latent(
        pooled, eps,
        p["agg_W"], p["agg_b"].reshape(1, 256),
        p["mu_W"], p["mu_b"].reshape(1, 256),
        p["var_W"], p["var_b"].reshape(1, 256),
        p["dft_W"], p["dft_b"].reshape(1, 1536),
    )

    y2 = _dec_a(dft.reshape(N, 256), dis, p["d1_W"])
    (y2,) = _gcn_layer(y2, srcg, dstg, dis, p["d1_b"].reshape(1, 256),
                       wn=p["d2_W"], want_h=False)
    (y2,) = _gcn_layer(y2, srcg, dstg, dis, p["d2_b"].reshape(1, 256),
                       wn=p["d3_W"], want_h=False)
    accd3 = _gcn_sc(y2.reshape(2 * NP, 128), srcg, dstg).reshape(2, NP, 128)
    out = _heads(accd3, dis, p["d3_b"].reshape(1, 256), w0cat, b0cat, bdp, b1p)

    pos_out = out[:, 0:2]
    acttype = out[:, 2:8]
    direc = out[:, 8:9]
    laneidx = out[:, 9:19]
    return (pos_out, acttype, direc, laneidx, lv, mu)


# speedup vs baseline: 8.5418x; 1.0326x over previous
import functools

import jax
import jax.numpy as jnp
from jax import lax
from jax.experimental import pallas as pl
from jax.experimental.pallas import tpu as pltpu
from jax.experimental.pallas import tpu_sc as plsc

N = 12000
E = 192000
G = 2000
NP = 12288

NC, NS = 2, 16

_MESH = plsc.VectorSubcoreMesh(
    core_axis_name="c", subcore_axis_name="s", num_cores=NC, num_subcores=NS
)


ROWS_PER_SUB = NP // NS
EDGES_PER_TILE = E // NS
GCN_CHUNK = 60
GCN_BLK = 25
GCN_NBLK = 8


@functools.partial(
    pl.kernel,
    out_type=jax.ShapeDtypeStruct((2 * NP, 128), jnp.float32),
    mesh=_MESH,
    scratch_types=[
        pltpu.VMEM((GCN_BLK, GCN_CHUNK), jnp.int32),
        pltpu.VMEM((GCN_BLK, GCN_CHUNK), jnp.int32),
        pltpu.VMEM((3, GCN_CHUNK, 128), jnp.float32),
        pltpu.VMEM_SHARED((NP, 128), jnp.float32),
        pltpu.SemaphoreType.DMA((3,)),
        pltpu.SemaphoreType.DMA((3,)),
    ],
)
def _gcn_sc(table, src, dst, out, sidx2, didx2, rows2, acc, sem_g, sem_s):
    core = lax.axis_index("c")
    sub = lax.axis_index("s")
    pltpu.sync_copy(
        table.at[pl.ds(core * NP + sub * ROWS_PER_SUB, ROWS_PER_SUB)],
        acc.at[pl.ds(sub * ROWS_PER_SUB, ROWS_PER_SUB)],
    )
    plsc.subcore_barrier()

    def block(b, carry0):
        pltpu.sync_copy(src.at[core, sub, b], sidx2)
        pltpu.sync_copy(dst.at[sub, b], didx2)
        pltpu.async_copy(table.at[sidx2.at[0]], rows2.at[0], sem_g.at[0])
        pltpu.async_copy(table.at[sidx2.at[1]], rows2.at[1], sem_g.at[1])

        def step(k, carry):
            slot = lax.rem(k, 3)
            pltpu.make_async_copy(
                table.at[sidx2.at[k]], rows2.at[slot], sem_g.at[slot]
            ).wait()
            pltpu.async_copy(
                rows2.at[slot], acc.at[didx2.at[k]], sem_s.at[slot], add=True
            )

            @pl.when(k + 2 < GCN_BLK)
            def _fire_next_gather():
                nslot = lax.rem(k + 2, 3)

                @pl.when(k > 0)
                def _wait_prev_scatter():
                    pltpu.make_async_copy(
                        rows2.at[nslot], acc.at[didx2.at[k - 1]], sem_s.at[nslot]
                    ).wait()

                pltpu.async_copy(
                    table.at[sidx2.at[k + 2]], rows2.at[nslot], sem_g.at[nslot]
                )

            return carry

        lax.fori_loop(0, GCN_BLK, step, 0)
        for t in (GCN_BLK - 3, GCN_BLK - 2, GCN_BLK - 1):
            pltpu.make_async_copy(
                rows2.at[t % 3], acc.at[didx2.at[t]], sem_s.at[t % 3]
            ).wait()
        return carry0

    lax.fori_loop(0, GCN_NBLK, block, 0)
    plsc.subcore_barrier()
    pltpu.sync_copy(
        acc.at[pl.ds(sub * ROWS_PER_SUB, ROWS_PER_SUB)],
        out.at[pl.ds(core * NP + sub * ROWS_PER_SUB, ROWS_PER_SUB)],
    )


DEG_PER_TILE = (E // 2) // NS
DEG_RPS = NP // NS


@functools.partial(
    pl.kernel,
    out_type=jax.ShapeDtypeStruct((2, NP), jnp.float32),
    mesh=_MESH,
    compiler_params=pltpu.CompilerParams(needs_layout_passes=False),
    scratch_types=[
        pltpu.VMEM((DEG_PER_TILE,), jnp.int32),
        pltpu.VMEM((NP,), jnp.float32),
        pltpu.VMEM((NS, DEG_RPS), jnp.float32),
        pltpu.VMEM((DEG_RPS,), jnp.float32),
        pltpu.VMEM_SHARED((NS, NP), jnp.float32),
    ],
)
def _deg_sc(dst, out, didx1, hist, colsum, acc1, shared):
    core = lax.axis_index("c")
    sub = lax.axis_index("s")
    pltpu.sync_copy(
        dst.at[pl.ds(core * (E // 2) + sub * DEG_PER_TILE, DEG_PER_TILE)],
        didx1,
    )
    zeros16 = jnp.zeros((16,), jnp.float32)
    ones16 = jnp.ones((16,), jnp.float32)

    def zstep(i, c):
        hist[pl.ds(i * 16, 16)] = zeros16
        return c

    lax.fori_loop(0, NP // 16, zstep, 0)

    def hstep(k, c):
        v = didx1[pl.ds(k * 16, 16)]
        plsc.addupdate_scatter(hist, [v], ones16)
        return c

    lax.fori_loop(0, DEG_PER_TILE // 16, hstep, 0)
    pltpu.sync_copy(hist, shared.at[sub])
    plsc.subcore_barrier()
    pltpu.sync_copy(shared.at[:, pl.ds(sub * DEG_RPS, DEG_RPS)], colsum)

    def rstep(j, c):
        s = colsum[0, pl.ds(j * 16, 16)]
        for r in range(1, NS):
            s = s + colsum[r, pl.ds(j * 16, 16)]
        acc1[pl.ds(j * 16, 16)] = s
        return c

    lax.fori_loop(0, DEG_RPS // 16, rstep, 0)
    pltpu.sync_copy(acc1, out.at[core, pl.ds(sub * DEG_RPS, DEG_RPS)])


POOL_ROWS = 2048
POOL_ITEMS = 12288
POOL_PER_TILE = POOL_ITEMS // NS
POOL_CHUNK = 96
POOL_NCHUNK = POOL_PER_TILE // POOL_CHUNK
POOL_RPS = POOL_ROWS // NS
POOL_D = 128
POOL_SLABS = 12


@functools.partial(
    pl.kernel,
    out_type=jax.ShapeDtypeStruct((POOL_SLABS, POOL_ROWS, POOL_D), jnp.float32),
    mesh=_MESH,
    scratch_types=[
        pltpu.VMEM((POOL_NCHUNK, POOL_CHUNK), jnp.int32),
        pltpu.VMEM((POOL_NCHUNK, POOL_CHUNK), jnp.int32),
        pltpu.VMEM((2, POOL_CHUNK, POOL_D), jnp.float32),
        pltpu.VMEM_SHARED((POOL_ROWS, POOL_D), jnp.float32),
        pltpu.SemaphoreType.DMA((2,)),
        pltpu.SemaphoreType.DMA((2,)),
    ],
)
def _pool_sc(table, src, dst, zinit, out, sidx2, didx2, rows2, acc, sem_g, sem_s):
    core = lax.axis_index("c")
    sub = lax.axis_index("s")
    pltpu.sync_copy(dst.at[sub], didx2)

    def ppass(p, carry0):
        h = core * (POOL_SLABS // 2) + p
        pltpu.sync_copy(src.at[h, sub], sidx2)
        pltpu.sync_copy(zinit, acc.at[pl.ds(sub * POOL_RPS, POOL_RPS)])
        plsc.subcore_barrier()

        pltpu.async_copy(table.at[sidx2.at[0]], rows2.at[0], sem_g.at[0])

        def step(k, carry):
            slot = lax.rem(k, 2)
            nslot = lax.rem(k + 1, 2)

            @pl.when(k > 0)
            def _wait_prev_scatter():
                pltpu.make_async_copy(
                    rows2.at[nslot], acc.at[didx2.at[k - 1]], sem_s.at[nslot]
                ).wait()

            @pl.when(k + 1 < POOL_NCHUNK)
            def _fire_next_gather():
                pltpu.async_copy(
                    table.at[sidx2.at[k + 1]], rows2.at[nslot], sem_g.at[nslot]
                )

            pltpu.make_async_copy(
                table.at[sidx2.at[k]], rows2.at[slot], sem_g.at[slot]
            ).wait()
            pltpu.async_copy(
                rows2.at[slot], acc.at[didx2.at[k]], sem_s.at[slot], add=True
            )
            return carry

        lax.fori_loop(0, POOL_NCHUNK, step, 0)
        pltpu.make_async_copy(
            rows2.at[(POOL_NCHUNK - 1) % 2],
            acc.at[didx2.at[POOL_NCHUNK - 1]],
            sem_s.at[(POOL_NCHUNK - 1) % 2],
        ).wait()
        plsc.subcore_barrier()
        pltpu.sync_copy(
            acc.at[pl.ds(sub * POOL_RPS, POOL_RPS)],
            out.at[h, pl.ds(sub * POOL_RPS, POOL_RPS)],
        )
        plsc.subcore_barrier()
        return carry0

    lax.fori_loop(0, POOL_SLABS // 2, ppass, 0)


BM = 512
NB = (N + BM - 1) // BM
NBG = 4


def _finalize_deg(degfull):
    def body(d_ref, o_ref):
        s = d_ref[0] + d_ref[1] + 1.0
        o_ref[...] = jnp.broadcast_to(
            lax.rsqrt(s).reshape(NP, 1), (NP, 128)
        )

    return pl.pallas_call(
        body,
        out_shape=jax.ShapeDtypeStruct((NP, 128), jnp.float32),
        grid=(1,),
        in_specs=[pl.BlockSpec((2, NP), lambda i: (0, 0))],
        out_specs=pl.BlockSpec((NP, 128), lambda i: (0, 0)),
    )(degfull)


def _enc_a1(x19, dis, wenc, benc, w1):
    def body(x_ref, dis_ref, we_ref, be_ref, w1_ref, n0_ref, y2_ref):
        n0 = jnp.maximum(
            jnp.dot(x_ref[...], we_ref[...], preferred_element_type=jnp.float32)
            + be_ref[...],
            0.0,
        )
        n0_ref[...] = n0
        d = dis_ref[:, 0:1]
        y = jnp.dot(n0, w1_ref[...], preferred_element_type=jnp.float32) * d
        y2_ref[0] = y[:, :128]
        y2_ref[1] = y[:, 128:]

    return pl.pallas_call(
        body,
        out_shape=(
            jax.ShapeDtypeStruct((N, 512), jnp.float32),
            jax.ShapeDtypeStruct((2, NP, 128), jnp.float32),
        ),
        grid=(NB,),
        in_specs=[
            pl.BlockSpec((BM, 19), lambda i: (i, 0)),
            pl.BlockSpec((BM, 128), lambda i: (i, 0)),
            pl.BlockSpec((19, 512), lambda i: (0, 0)),
            pl.BlockSpec((1, 512), lambda i: (0, 0)),
            pl.BlockSpec((512, 256), lambda i: (0, 0)),
        ],
        out_specs=(
            pl.BlockSpec((BM, 512), lambda i: (i, 0)),
            pl.BlockSpec((2, BM, 128), lambda i: (0, i, 0)),
        ),
    )(x19, dis, wenc, benc, w1)


def _gcn_post(acc2, dis, b, wn=None, want_h=False):
    def body(*refs):
        if wn is not None:
            a_ref, dis_ref, b_ref, wn_ref = refs[:4]
            orefs = refs[4:]
        else:
            a_ref, dis_ref, b_ref = refs[:3]
            orefs = refs[3:]
        d = dis_ref[:, 0:1]
        a = jnp.concatenate([a_ref[0], a_ref[1]], axis=1)
        h = jnp.maximum(a * d + b_ref[...], 0.0)
        i = 0
        if want_h:
            orefs[i][...] = h
            i += 1
        if wn is not None:
            y = jnp.dot(h, wn_ref[...], preferred_element_type=jnp.float32) * d
            orefs[i][0] = y[:, :128]
            orefs[i][1] = y[:, 128:]

    in_specs = [
        pl.BlockSpec((2, BM, 128), lambda i: (0, i, 0)),
        pl.BlockSpec((BM, 128), lambda i: (i, 0)),
        pl.BlockSpec((1, 256), lambda i: (0, 0)),
    ]
    args = [acc2, dis, b]
    if wn is not None:
        in_specs.append(pl.BlockSpec((256, 256), lambda i: (0, 0)))
        args.append(wn)
    out_shape, out_specs = [], []
    if want_h:
        out_shape.append(jax.ShapeDtypeStruct((N, 256), jnp.float32))
        out_specs.append(pl.BlockSpec((BM, 256), lambda i: (i, 0)))
    if wn is not None:
        out_shape.append(jax.ShapeDtypeStruct((2, NP, 128), jnp.float32))
        out_specs.append(pl.BlockSpec((2, BM, 128), lambda i: (0, i, 0)))
    return pl.pallas_call(
        body,
        out_shape=tuple(out_shape),
        grid=(NB,),
        in_specs=in_specs,
        out_specs=tuple(out_specs),
    )(*args)


def _latent(pooled, eps, aggw, aggb, muw, mub, varw, varb, dftw, dftb):
    def body(p_ref, e_ref, aw, ab, mw, mb, vw, vb, dw, db,
             mu_ref, lv_ref, dft_ref):
        cnt = p_ref[10][:, 0:1]
        g = jnp.concatenate(
            [p_ref[h] for h in range(10)], axis=1
        ) / jnp.maximum(cnt, 1.0)
        latent = jnp.dot(g, aw[...], preferred_element_type=jnp.float32) + ab[...]
        mu = jnp.dot(latent, mw[...], preferred_element_type=jnp.float32) + mb[...]
        lv = jnp.dot(latent, vw[...], preferred_element_type=jnp.float32) + vb[...]
        mu_ref[...] = mu
        lv_ref[...] = lv
        z = e_ref[...] * jnp.exp(0.5 * lv) + mu
        dft_ref[...] = jnp.dot(z, dw[...], preferred_element_type=jnp.float32) + db[...]

    return pl.pallas_call(
        body,
        out_shape=(
            jax.ShapeDtypeStruct((G, 256), jnp.float32),
            jax.ShapeDtypeStruct((G, 256), jnp.float32),
            jax.ShapeDtypeStruct((G, 1536), jnp.float32),
        ),
        grid=(NBG,),
        in_specs=[
            pl.BlockSpec((POOL_SLABS, BM, POOL_D), lambda i: (0, i, 0)),
            pl.BlockSpec((BM, 256), lambda i: (i, 0)),
            pl.BlockSpec((1280, 256), lambda i: (0, 0)),
            pl.BlockSpec((1, 256), lambda i: (0, 0)),
            pl.BlockSpec((256, 256), lambda i: (0, 0)),
            pl.BlockSpec((1, 256), lambda i: (0, 0)),
            pl.BlockSpec((256, 256), lambda i: (0, 0)),
            pl.BlockSpec((1, 256), lambda i: (0, 0)),
            pl.BlockSpec((256, 1536), lambda i: (0, 0)),
            pl.BlockSpec((1, 1536), lambda i: (0, 0)),
        ],
        out_specs=(
            pl.BlockSpec((BM, 256), lambda i: (i, 0)),
            pl.BlockSpec((BM, 256), lambda i: (i, 0)),
            pl.BlockSpec((BM, 1536), lambda i: (i, 0)),
        ),
    )(pooled, eps, aggw, aggb, muw, mub, varw, varb, dftw, dftb)


def _dec_a(z6, dis, w):
    def body(z_ref, dis_ref, w_ref, y2_ref):
        d = dis_ref[:, 0:1]
        h = jnp.maximum(z_ref[...], 0.0)
        y = jnp.dot(h, w_ref[...], preferred_element_type=jnp.float32) * d
        y2_ref[0] = y[:, :128]
        y2_ref[1] = y[:, 128:]

    return pl.pallas_call(
        body,
        out_shape=jax.ShapeDtypeStruct((2, NP, 128), jnp.float32),
        grid=(NB,),
        in_specs=[
            pl.BlockSpec((BM, 256), lambda i: (i, 0)),
            pl.BlockSpec((BM, 128), lambda i: (i, 0)),
            pl.BlockSpec((256, 256), lambda i: (0, 0)),
        ],
        out_specs=pl.BlockSpec((2, BM, 128), lambda i: (0, i, 0)),
    )(z6, dis, w)


def _heads(acc2, dis, b3, w0cat, b0cat, bdp, b1p):
    def body(a_ref, dis_ref, b3_ref, w0_ref, b0_ref, bd_ref, b1_ref, o_ref):
        d = dis_ref[:, 0:1]
        a = jnp.concatenate([a_ref[0], a_ref[1]], axis=1)
        h3 = jnp.maximum(a * d + b3_ref[...], 0.0)
        hh = jnp.maximum(
            jnp.dot(h3, w0_ref[...], preferred_element_type=jnp.float32)
            + b0_ref[...],
            0.0,
        )
        o_ref[...] = (
            jnp.dot(hh, bd_ref[...], preferred_element_type=jnp.float32)
            + b1_ref[...]
        )

    return pl.pallas_call(
        body,
        out_shape=jax.ShapeDtypeStruct((N, 128), jnp.float32),
        grid=(NB,),
        in_specs=[
            pl.BlockSpec((2, BM, 128), lambda i: (0, i, 0)),
            pl.BlockSpec((BM, 128), lambda i: (i, 0)),
            pl.BlockSpec((1, 256), lambda i: (0, 0)),
            pl.BlockSpec((256, 1280), lambda i: (0, 0)),
            pl.BlockSpec((1, 1280), lambda i: (0, 0)),
            pl.BlockSpec((1280, 128), lambda i: (0, 0)),
            pl.BlockSpec((1, 128), lambda i: (0, 0)),
        ],
        out_specs=pl.BlockSpec((BM, 128), lambda i: (i, 0)),
    )(acc2, dis, b3, w0cat, b0cat, bdp, b1p)


def _gcn_layer(y2, srcg, dstg, dis, b, wn=None, want_h=False):
    acc = _gcn_sc(y2.reshape(2 * NP, 128), srcg, dstg)
    return _gcn_post(acc.reshape(2, NP, 128), dis, b, wn=wn, want_h=want_h)


def kernel(pos, actor_type, lane_index, direction, params, edge_index, batch):
    p = params
    f32 = jnp.float32
    row, col = edge_index[0], edge_index[1]
    srcg = jnp.concatenate([row, row + NP]).reshape(
        2, NS, GCN_NBLK, GCN_BLK, GCN_CHUNK
    )
    dstg = col.reshape(NS, GCN_NBLK, GCN_BLK, GCN_CHUNK)

    wenc = jnp.zeros((19, 512), f32)
    wenc = wenc.at[0:2, 0:128].set(p["pos_W"])
    wenc = wenc.at[2:8, 128:256].set(p["type_W"])
    wenc = wenc.at[8:18, 256:384].set(p["lane_W"])
    wenc = wenc.at[18:19, 384:512].set(p["dir_W"])
    benc = jnp.concatenate(
        [p["pos_b"], p["type_b"], p["lane_b"], p["dir_b"]]
    ).reshape(1, 512)
    x19 = jnp.concatenate([pos, actor_type, lane_index, direction], axis=1)

    w0cat = jnp.concatenate(
        [p["px0_W"], p["py0_W"], p["at0_W"], p["dr0_W"], p["li0_W"]], axis=1
    )
    b0cat = jnp.concatenate(
        [p["px0_b"], p["py0_b"], p["at0_b"], p["dr0_b"], p["li0_b"]]
    ).reshape(1, 1280)
    bdp = jnp.zeros((1280, 128), f32)
    bdp = bdp.at[0:256, 0:1].set(p["px1_W"])
    bdp = bdp.at[256:512, 1:2].set(p["py1_W"])
    bdp = bdp.at[512:768, 2:8].set(p["at1_W"])
    bdp = bdp.at[768:1024, 8:9].set(p["dr1_W"])
    bdp = bdp.at[1024:1280, 9:19].set(p["li1_W"])
    b1p = jnp.zeros((128,), f32)
    b1p = b1p.at[0:1].set(p["px1_b"])
    b1p = b1p.at[1:2].set(p["py1_b"])
    b1p = b1p.at[2:8].set(p["at1_b"])
    b1p = b1p.at[8:9].set(p["dr1_b"])
    b1p = b1p.at[9:19].set(p["li1_b"])
    b1p = b1p.reshape(1, 128)

    eps = jax.random.normal(jax.random.key(42), (G, 256), dtype=f32)

    degfull = _deg_sc(col)
    dis = _finalize_deg(degfull)

    n0, y2 = _enc_a1(x19, dis, wenc, benc, p["e1_W"])
    h1, y2 = _gcn_layer(y2, srcg, dstg, dis, p["e1_b"].reshape(1, 256),
                        wn=p["e2_W"], want_h=True)
    h2, y2 = _gcn_layer(y2, srcg, dstg, dis, p["e2_b"].reshape(1, 256),
                        wn=p["e3_W"], want_h=True)
    (h3,) = _gcn_layer(y2, srcg, dstg, dis, p["e3_b"].reshape(1, 256),
                       wn=None, want_h=True)

    xcat = jnp.concatenate(
        [n0, h1, h2, h3, jnp.ones((N, 256), f32)], axis=1
    )
    ptab = jnp.concatenate(
        [xcat[:, h * POOL_D:(h + 1) * POOL_D] for h in range(POOL_SLABS)],
        axis=0,
    )
    ar = jnp.arange(N, dtype=jnp.int32)
    pad_src = jnp.zeros((POOL_ITEMS - N,), jnp.int32)
    srcp_half = jnp.concatenate([ar, pad_src])
    srcp = jnp.concatenate(
        [srcp_half + h * N for h in range(POOL_SLABS)]
    ).reshape(POOL_SLABS, NS, POOL_NCHUNK, POOL_CHUNK)
    batch_pad = jnp.concatenate(
        [batch, jnp.full((POOL_ITEMS - N,), G, jnp.int32)]
    ).reshape(NS, POOL_NCHUNK, POOL_CHUNK)
    pooled = _pool_sc(ptab, srcp, batch_pad, jnp.zeros((POOL_RPS, POOL_D), f32))

    mu, lv, dft = _latent(
        pooled, eps,
        p["agg_W"], p["agg_b"].reshape(1, 256),
        p["mu_W"], p["mu_b"].reshape(1, 256),
        p["var_W"], p["var_b"].reshape(1, 256),
        p["dft_W"], p["dft_b"].reshape(1, 1536),
    )

    y2 = _dec_a(dft.reshape(N, 256), dis, p["d1_W"])
    (y2,) = _gcn_layer(y2, srcg, dstg, dis, p["d1_b"].reshape(1, 256),
                       wn=p["d2_W"], want_h=False)
    (y2,) = _gcn_layer(y2, srcg, dstg, dis, p["d2_b"].reshape(1, 256),
                       wn=p["d3_W"], want_h=False)
    accd3 = _gcn_sc(y2.reshape(2 * NP, 128), srcg, dstg).reshape(2, NP, 128)
    out = _heads(accd3, dis, p["d3_b"].reshape(1, 256), w0cat, b0cat, bdp, b1p)

    pos_out = out[:, 0:2]
    acttype = out[:, 2:8]
    direc = out[:, 8:9]
    laneidx = out[:, 9:19]
    return (pos_out, acttype, direc, laneidx, lv, mu)


# speedup vs baseline: 8.9404x; 1.0467x over previous
import functools

import jax
import jax.numpy as jnp
from jax import lax
from jax.experimental import pallas as pl
from jax.experimental.pallas import tpu as pltpu
from jax.experimental.pallas import tpu_sc as plsc

N = 12000
E = 192000
G = 2000
NP = 12288

NC, NS = 2, 16

_MESH = plsc.VectorSubcoreMesh(
    core_axis_name="c", subcore_axis_name="s", num_cores=NC, num_subcores=NS
)


ROWS_PER_SUB = NP // NS
EDGES_PER_TILE = E // NS
GCN_CHUNK = 60
GCN_BLK = 25
GCN_NBLK = 8


@functools.partial(
    pl.kernel,
    out_type=jax.ShapeDtypeStruct((2 * NP, 128), jnp.float32),
    mesh=_MESH,
    scratch_types=[
        pltpu.VMEM((GCN_BLK, GCN_CHUNK), jnp.int32),
        pltpu.VMEM((GCN_BLK, GCN_CHUNK), jnp.int32),
        pltpu.VMEM((3, GCN_CHUNK, 128), jnp.float32),
        pltpu.VMEM_SHARED((NP, 128), jnp.float32),
        pltpu.SemaphoreType.DMA((3,)),
        pltpu.SemaphoreType.DMA((3,)),
    ],
)
def _gcn_sc(table, src, dst, out, sidx2, didx2, rows2, acc, sem_g, sem_s):
    core = lax.axis_index("c")
    sub = lax.axis_index("s")
    pltpu.sync_copy(
        table.at[pl.ds(core * NP + sub * ROWS_PER_SUB, ROWS_PER_SUB)],
        acc.at[pl.ds(sub * ROWS_PER_SUB, ROWS_PER_SUB)],
    )
    plsc.subcore_barrier()

    def block(b, carry0):
        pltpu.sync_copy(src.at[core, sub, b], sidx2)
        pltpu.sync_copy(dst.at[sub, b], didx2)
        pltpu.async_copy(table.at[sidx2.at[0]], rows2.at[0], sem_g.at[0])
        pltpu.async_copy(table.at[sidx2.at[1]], rows2.at[1], sem_g.at[1])

        def step(k, carry):
            slot = lax.rem(k, 3)
            pltpu.make_async_copy(
                table.at[sidx2.at[k]], rows2.at[slot], sem_g.at[slot]
            ).wait()
            pltpu.async_copy(
                rows2.at[slot], acc.at[didx2.at[k]], sem_s.at[slot], add=True
            )

            @pl.when(k + 2 < GCN_BLK)
            def _fire_next_gather():
                nslot = lax.rem(k + 2, 3)

                @pl.when(k > 0)
                def _wait_prev_scatter():
                    pltpu.make_async_copy(
                        rows2.at[nslot], acc.at[didx2.at[k - 1]], sem_s.at[nslot]
                    ).wait()

                pltpu.async_copy(
                    table.at[sidx2.at[k + 2]], rows2.at[nslot], sem_g.at[nslot]
                )

            return carry

        lax.fori_loop(0, GCN_BLK, step, 0)
        for t in (GCN_BLK - 3, GCN_BLK - 2, GCN_BLK - 1):
            pltpu.make_async_copy(
                rows2.at[t % 3], acc.at[didx2.at[t]], sem_s.at[t % 3]
            ).wait()
        return carry0

    lax.fori_loop(0, GCN_NBLK, block, 0)
    plsc.subcore_barrier()
    pltpu.sync_copy(
        acc.at[pl.ds(sub * ROWS_PER_SUB, ROWS_PER_SUB)],
        out.at[pl.ds(core * NP + sub * ROWS_PER_SUB, ROWS_PER_SUB)],
    )


DEG_PER_TILE = (E // 2) // NS
DEG_RPS = NP // NS


CNT_BINS = 2048
CNT_PER_TILE = 12288 // 2 // NS
CNT_RPS = CNT_BINS // NS


@functools.partial(
    pl.kernel,
    out_type=(
        jax.ShapeDtypeStruct((2, NP), jnp.float32),
        jax.ShapeDtypeStruct((2, CNT_BINS), jnp.float32),
    ),
    mesh=_MESH,
    compiler_params=pltpu.CompilerParams(needs_layout_passes=False),
    scratch_types=[
        pltpu.VMEM((DEG_PER_TILE,), jnp.int32),
        pltpu.VMEM((CNT_PER_TILE,), jnp.int32),
        pltpu.VMEM((NP,), jnp.float32),
        pltpu.VMEM((CNT_BINS,), jnp.float32),
        pltpu.VMEM((NS, DEG_RPS), jnp.float32),
        pltpu.VMEM((DEG_RPS,), jnp.float32),
        pltpu.VMEM_SHARED((NS, NP), jnp.float32),
        pltpu.VMEM_SHARED((NS, CNT_BINS), jnp.float32),
    ],
)
def _deg_sc(dst, bat, out, outc, didx1, bidx1, hist, histc, colsum, acc1,
            shared, sharedc):
    core = lax.axis_index("c")
    sub = lax.axis_index("s")
    pltpu.sync_copy(
        dst.at[pl.ds(core * (E // 2) + sub * DEG_PER_TILE, DEG_PER_TILE)],
        didx1,
    )
    pltpu.sync_copy(
        bat.at[pl.ds(core * (12288 // 2) + sub * CNT_PER_TILE, CNT_PER_TILE)],
        bidx1,
    )
    zeros16 = jnp.zeros((16,), jnp.float32)
    ones16 = jnp.ones((16,), jnp.float32)

    def zstep(i, c):
        hist[pl.ds(i * 16, 16)] = zeros16
        return c

    lax.fori_loop(0, NP // 16, zstep, 0)

    def zstepc(i, c):
        histc[pl.ds(i * 16, 16)] = zeros16
        return c

    lax.fori_loop(0, CNT_BINS // 16, zstepc, 0)

    def hstep(k, c):
        v = didx1[pl.ds(k * 16, 16)]
        plsc.addupdate_scatter(hist, [v], ones16)
        return c

    lax.fori_loop(0, DEG_PER_TILE // 16, hstep, 0)

    def hstepc(k, c):
        v = bidx1[pl.ds(k * 16, 16)]
        plsc.addupdate_scatter(histc, [v], ones16)
        return c

    lax.fori_loop(0, CNT_PER_TILE // 16, hstepc, 0)
    pltpu.sync_copy(hist, shared.at[sub])
    pltpu.sync_copy(histc, sharedc.at[sub])
    plsc.subcore_barrier()
    pltpu.sync_copy(shared.at[:, pl.ds(sub * DEG_RPS, DEG_RPS)], colsum)

    def rstep(j, c):
        s = colsum[0, pl.ds(j * 16, 16)]
        for r in range(1, NS):
            s = s + colsum[r, pl.ds(j * 16, 16)]
        acc1[pl.ds(j * 16, 16)] = s
        return c

    lax.fori_loop(0, DEG_RPS // 16, rstep, 0)
    pltpu.sync_copy(acc1, out.at[core, pl.ds(sub * DEG_RPS, DEG_RPS)])
    pltpu.sync_copy(sharedc.at[:, pl.ds(sub * CNT_RPS, CNT_RPS)], colsum.at[:, pl.ds(0, CNT_RPS)])

    def rstepc(j, c):
        s = colsum[0, pl.ds(j * 16, 16)]
        for r in range(1, NS):
            s = s + colsum[r, pl.ds(j * 16, 16)]
        acc1[pl.ds(j * 16, 16)] = s
        return c

    lax.fori_loop(0, CNT_RPS // 16, rstepc, 0)
    pltpu.sync_copy(
        acc1.at[pl.ds(0, CNT_RPS)], outc.at[core, pl.ds(sub * CNT_RPS, CNT_RPS)]
    )


POOL_ROWS = 2048
POOL_ITEMS = 12288
POOL_PER_TILE = POOL_ITEMS // NS
POOL_CHUNK = 96
POOL_NCHUNK = POOL_PER_TILE // POOL_CHUNK
POOL_RPS = POOL_ROWS // NS
POOL_D = 128
POOL_SLABS = 10


@functools.partial(
    pl.kernel,
    out_type=jax.ShapeDtypeStruct((POOL_SLABS, POOL_ROWS, POOL_D), jnp.float32),
    mesh=_MESH,
    scratch_types=[
        pltpu.VMEM((POOL_NCHUNK, POOL_CHUNK), jnp.int32),
        pltpu.VMEM((POOL_NCHUNK, POOL_CHUNK), jnp.int32),
        pltpu.VMEM((2, POOL_CHUNK, POOL_D), jnp.float32),
        pltpu.VMEM_SHARED((POOL_ROWS, POOL_D), jnp.float32),
        pltpu.SemaphoreType.DMA((2,)),
        pltpu.SemaphoreType.DMA((2,)),
    ],
)
def _pool_sc(table, src, dst, zinit, out, sidx2, didx2, rows2, acc, sem_g, sem_s):
    core = lax.axis_index("c")
    sub = lax.axis_index("s")
    pltpu.sync_copy(dst.at[sub], didx2)

    def ppass(p, carry0):
        h = core * (POOL_SLABS // 2) + p
        pltpu.sync_copy(src.at[h, sub], sidx2)
        pltpu.sync_copy(zinit, acc.at[pl.ds(sub * POOL_RPS, POOL_RPS)])
        plsc.subcore_barrier()

        pltpu.async_copy(table.at[sidx2.at[0]], rows2.at[0], sem_g.at[0])

        def step(k, carry):
            slot = lax.rem(k, 2)
            nslot = lax.rem(k + 1, 2)

            @pl.when(k > 0)
            def _wait_prev_scatter():
                pltpu.make_async_copy(
                    rows2.at[nslot], acc.at[didx2.at[k - 1]], sem_s.at[nslot]
                ).wait()

            @pl.when(k + 1 < POOL_NCHUNK)
            def _fire_next_gather():
                pltpu.async_copy(
                    table.at[sidx2.at[k + 1]], rows2.at[nslot], sem_g.at[nslot]
                )

            pltpu.make_async_copy(
                table.at[sidx2.at[k]], rows2.at[slot], sem_g.at[slot]
            ).wait()
            pltpu.async_copy(
                rows2.at[slot], acc.at[didx2.at[k]], sem_s.at[slot], add=True
            )
            return carry

        lax.fori_loop(0, POOL_NCHUNK, step, 0)
        pltpu.make_async_copy(
            rows2.at[(POOL_NCHUNK - 1) % 2],
            acc.at[didx2.at[POOL_NCHUNK - 1]],
            sem_s.at[(POOL_NCHUNK - 1) % 2],
        ).wait()
        plsc.subcore_barrier()
        pltpu.sync_copy(
            acc.at[pl.ds(sub * POOL_RPS, POOL_RPS)],
            out.at[h, pl.ds(sub * POOL_RPS, POOL_RPS)],
        )
        plsc.subcore_barrier()
        return carry0

    lax.fori_loop(0, POOL_SLABS // 2, ppass, 0)


BM = 512
NB = (N + BM - 1) // BM
NBG = 4


def _finalize_deg(degfull):
    def body(d_ref, o_ref):
        s = d_ref[0] + d_ref[1] + 1.0
        o_ref[...] = jnp.broadcast_to(
            lax.rsqrt(s).reshape(NP, 1), (NP, 128)
        )

    return pl.pallas_call(
        body,
        out_shape=jax.ShapeDtypeStruct((NP, 128), jnp.float32),
        grid=(1,),
        in_specs=[pl.BlockSpec((2, NP), lambda i: (0, 0))],
        out_specs=pl.BlockSpec((NP, 128), lambda i: (0, 0)),
    )(degfull)


def _enc_a1(x19, dis, wenc, benc, w1):
    def body(x_ref, dis_ref, we_ref, be_ref, w1_ref, pt_ref, y2_ref):
        n0 = jnp.maximum(
            jnp.dot(x_ref[...], we_ref[...], preferred_element_type=jnp.float32)
            + be_ref[...],
            0.0,
        )
        for j in range(4):
            pt_ref[j] = n0[:, j * 128:(j + 1) * 128]
        d = dis_ref[:, 0:1]
        y = jnp.dot(n0, w1_ref[...], preferred_element_type=jnp.float32) * d
        y2_ref[0] = y[:, :128]
        y2_ref[1] = y[:, 128:]

    return pl.pallas_call(
        body,
        out_shape=(
            jax.ShapeDtypeStruct((POOL_SLABS, N, 128), jnp.float32),
            jax.ShapeDtypeStruct((2, NP, 128), jnp.float32),
        ),
        grid=(NB,),
        in_specs=[
            pl.BlockSpec((BM, 19), lambda i: (i, 0)),
            pl.BlockSpec((BM, 128), lambda i: (i, 0)),
            pl.BlockSpec((19, 512), lambda i: (0, 0)),
            pl.BlockSpec((1, 512), lambda i: (0, 0)),
            pl.BlockSpec((512, 256), lambda i: (0, 0)),
        ],
        out_specs=(
            pl.BlockSpec((4, BM, 128), lambda i: (0, i, 0)),
            pl.BlockSpec((2, BM, 128), lambda i: (0, i, 0)),
        ),
    )(x19, dis, wenc, benc, w1)


def _gcn_post(acc2, dis, b, wn=None, ptab=None, slab=0):
    def body(*refs):
        n_in = 3 + (wn is not None) + (ptab is not None)
        a_ref, dis_ref, b_ref = refs[:3]
        wn_ref = refs[3] if wn is not None else None
        orefs = refs[n_in:]
        d = dis_ref[:, 0:1]
        a = jnp.concatenate([a_ref[0], a_ref[1]], axis=1)
        h = jnp.maximum(a * d + b_ref[...], 0.0)
        i = 0
        if ptab is not None:
            orefs[i][0] = h[:, :128]
            orefs[i][1] = h[:, 128:]
            i += 1
        if wn is not None:
            y = jnp.dot(h, wn_ref[...], preferred_element_type=jnp.float32) * d
            orefs[i][0] = y[:, :128]
            orefs[i][1] = y[:, 128:]

    in_specs = [
        pl.BlockSpec((2, BM, 128), lambda i: (0, i, 0)),
        pl.BlockSpec((BM, 128), lambda i: (i, 0)),
        pl.BlockSpec((1, 256), lambda i: (0, 0)),
    ]
    args = [acc2, dis, b]
    if wn is not None:
        in_specs.append(pl.BlockSpec((256, 256), lambda i: (0, 0)))
        args.append(wn)
    aliases = {}
    out_shape, out_specs = [], []
    if ptab is not None:
        aliases[len(args)] = 0
        in_specs.append(pl.BlockSpec(memory_space=pl.ANY))
        args.append(ptab)
        out_shape.append(
            jax.ShapeDtypeStruct((POOL_SLABS, N, 128), jnp.float32)
        )
        sb = slab // 2
        out_specs.append(pl.BlockSpec((2, BM, 128), lambda i: (sb, i, 0)))
    if wn is not None:
        out_shape.append(jax.ShapeDtypeStruct((2, NP, 128), jnp.float32))
        out_specs.append(pl.BlockSpec((2, BM, 128), lambda i: (0, i, 0)))
    return pl.pallas_call(
        body,
        out_shape=tuple(out_shape),
        grid=(NB,),
        in_specs=in_specs,
        out_specs=tuple(out_specs),
        input_output_aliases=aliases,
    )(*args)


def _latent(pooled, cnt2, eps, aggw, aggb, muw, mub, varw, varb, dftw, dftb):
    def body(p_ref, c_ref, e_ref, aw, ab, mw, mb, vw, vb, dw, db,
             mu_ref, lv_ref, dft_ref):
        cnt = (c_ref[0] + c_ref[1]).reshape(c_ref.shape[1], 1)
        g = jnp.concatenate(
            [p_ref[h] for h in range(10)], axis=1
        ) / jnp.maximum(cnt, 1.0)
        latent = jnp.dot(g, aw[...], preferred_element_type=jnp.float32) + ab[...]
        mu = jnp.dot(latent, mw[...], preferred_element_type=jnp.float32) + mb[...]
        lv = jnp.dot(latent, vw[...], preferred_element_type=jnp.float32) + vb[...]
        mu_ref[...] = mu
        lv_ref[...] = lv
        z = e_ref[...] * jnp.exp(0.5 * lv) + mu
        dft_ref[...] = jnp.dot(z, dw[...], preferred_element_type=jnp.float32) + db[...]

    return pl.pallas_call(
        body,
        out_shape=(
            jax.ShapeDtypeStruct((G, 256), jnp.float32),
            jax.ShapeDtypeStruct((G, 256), jnp.float32),
            jax.ShapeDtypeStruct((G, 1536), jnp.float32),
        ),
        grid=(NBG,),
        in_specs=[
            pl.BlockSpec((POOL_SLABS, BM, POOL_D), lambda i: (0, i, 0)),
            pl.BlockSpec((2, BM), lambda i: (0, i)),
            pl.BlockSpec((BM, 256), lambda i: (i, 0)),
            pl.BlockSpec((1280, 256), lambda i: (0, 0)),
            pl.BlockSpec((1, 256), lambda i: (0, 0)),
            pl.BlockSpec((256, 256), lambda i: (0, 0)),
            pl.BlockSpec((1, 256), lambda i: (0, 0)),
            pl.BlockSpec((256, 256), lambda i: (0, 0)),
            pl.BlockSpec((1, 256), lambda i: (0, 0)),
            pl.BlockSpec((256, 1536), lambda i: (0, 0)),
            pl.BlockSpec((1, 1536), lambda i: (0, 0)),
        ],
        out_specs=(
            pl.BlockSpec((BM, 256), lambda i: (i, 0)),
            pl.BlockSpec((BM, 256), lambda i: (i, 0)),
            pl.BlockSpec((BM, 1536), lambda i: (i, 0)),
        ),
    )(pooled, cnt2, eps, aggw, aggb, muw, mub, varw, varb, dftw, dftb)


def _dec_a(z6, dis, w):
    def body(z_ref, dis_ref, w_ref, y2_ref):
        d = dis_ref[:, 0:1]
        h = jnp.maximum(z_ref[...], 0.0)
        y = jnp.dot(h, w_ref[...], preferred_element_type=jnp.float32) * d
        y2_ref[0] = y[:, :128]
        y2_ref[1] = y[:, 128:]

    return pl.pallas_call(
        body,
        out_shape=jax.ShapeDtypeStruct((2, NP, 128), jnp.float32),
        grid=(NB,),
        in_specs=[
            pl.BlockSpec((BM, 256), lambda i: (i, 0)),
            pl.BlockSpec((BM, 128), lambda i: (i, 0)),
            pl.BlockSpec((256, 256), lambda i: (0, 0)),
        ],
        out_specs=pl.BlockSpec((2, BM, 128), lambda i: (0, i, 0)),
    )(z6, dis, w)


def _heads(acc2, dis, b3, w0cat, b0cat, bdp, b1p):
    def body(a_ref, dis_ref, b3_ref, w0_ref, b0_ref, bd_ref, b1_ref, o_ref):
        d = dis_ref[:, 0:1]
        a = jnp.concatenate([a_ref[0], a_ref[1]], axis=1)
        h3 = jnp.maximum(a * d + b3_ref[...], 0.0)
        hh = jnp.maximum(
            jnp.dot(h3, w0_ref[...], preferred_element_type=jnp.float32)
            + b0_ref[...],
            0.0,
        )
        o_ref[...] = (
            jnp.dot(hh, bd_ref[...], preferred_element_type=jnp.float32)
            + b1_ref[...]
        )

    return pl.pallas_call(
        body,
        out_shape=jax.ShapeDtypeStruct((N, 128), jnp.float32),
        grid=(NB,),
        in_specs=[
            pl.BlockSpec((2, BM, 128), lambda i: (0, i, 0)),
            pl.BlockSpec((BM, 128), lambda i: (i, 0)),
            pl.BlockSpec((1, 256), lambda i: (0, 0)),
            pl.BlockSpec((256, 1280), lambda i: (0, 0)),
            pl.BlockSpec((1, 1280), lambda i: (0, 0)),
            pl.BlockSpec((1280, 128), lambda i: (0, 0)),
            pl.BlockSpec((1, 128), lambda i: (0, 0)),
        ],
        out_specs=pl.BlockSpec((BM, 128), lambda i: (i, 0)),
    )(acc2, dis, b3, w0cat, b0cat, bdp, b1p)


def _gcn_layer(y2, srcg, dstg, dis, b, wn=None, ptab=None, slab=0):
    acc = _gcn_sc(y2.reshape(2 * NP, 128), srcg, dstg)
    return _gcn_post(acc.reshape(2, NP, 128), dis, b, wn=wn, ptab=ptab,
                     slab=slab)


def kernel(pos, actor_type, lane_index, direction, params, edge_index, batch):
    p = params
    f32 = jnp.float32
    row, col = edge_index[0], edge_index[1]
    srcg = jnp.concatenate([row, row + NP]).reshape(
        2, NS, GCN_NBLK, GCN_BLK, GCN_CHUNK
    )
    dstg = col.reshape(NS, GCN_NBLK, GCN_BLK, GCN_CHUNK)

    wenc = jnp.zeros((19, 512), f32)
    wenc = wenc.at[0:2, 0:128].set(p["pos_W"])
    wenc = wenc.at[2:8, 128:256].set(p["type_W"])
    wenc = wenc.at[8:18, 256:384].set(p["lane_W"])
    wenc = wenc.at[18:19, 384:512].set(p["dir_W"])
    benc = jnp.concatenate(
        [p["pos_b"], p["type_b"], p["lane_b"], p["dir_b"]]
    ).reshape(1, 512)
    x19 = jnp.concatenate([pos, actor_type, lane_index, direction], axis=1)

    w0cat = jnp.concatenate(
        [p["px0_W"], p["py0_W"], p["at0_W"], p["dr0_W"], p["li0_W"]], axis=1
    )
    b0cat = jnp.concatenate(
        [p["px0_b"], p["py0_b"], p["at0_b"], p["dr0_b"], p["li0_b"]]
    ).reshape(1, 1280)
    bdp = jnp.zeros((1280, 128), f32)
    bdp = bdp.at[0:256, 0:1].set(p["px1_W"])
    bdp = bdp.at[256:512, 1:2].set(p["py1_W"])
    bdp = bdp.at[512:768, 2:8].set(p["at1_W"])
    bdp = bdp.at[768:1024, 8:9].set(p["dr1_W"])
    bdp = bdp.at[1024:1280, 9:19].set(p["li1_W"])
    b1p = jnp.zeros((128,), f32)
    b1p = b1p.at[0:1].set(p["px1_b"])
    b1p = b1p.at[1:2].set(p["py1_b"])
    b1p = b1p.at[2:8].set(p["at1_b"])
    b1p = b1p.at[8:9].set(p["dr1_b"])
    b1p = b1p.at[9:19].set(p["li1_b"])
    b1p = b1p.reshape(1, 128)

    eps = jax.random.normal(jax.random.key(42), (G, 256), dtype=f32)

    batch_pad = jnp.concatenate(
        [batch, jnp.full((POOL_ITEMS - N,), 2047, jnp.int32)]
    )

    degfull, cnt2 = _deg_sc(col, batch_pad)
    dis = _finalize_deg(degfull)

    ptab, y2 = _enc_a1(x19, dis, wenc, benc, p["e1_W"])
    (ptab, y2) = _gcn_layer(y2, srcg, dstg, dis, p["e1_b"].reshape(1, 256),
                            wn=p["e2_W"], ptab=ptab, slab=4)
    (ptab, y2) = _gcn_layer(y2, srcg, dstg, dis, p["e2_b"].reshape(1, 256),
                            wn=p["e3_W"], ptab=ptab, slab=6)
    (ptab,) = _gcn_layer(y2, srcg, dstg, dis, p["e3_b"].reshape(1, 256),
                         wn=None, ptab=ptab, slab=8)

    ar = jnp.arange(N, dtype=jnp.int32)
    pad_src = jnp.zeros((POOL_ITEMS - N,), jnp.int32)
    srcp_half = jnp.concatenate([ar, pad_src])
    srcp = jnp.concatenate(
        [srcp_half + h * N for h in range(POOL_SLABS)]
    ).reshape(POOL_SLABS, NS, POOL_NCHUNK, POOL_CHUNK)
    dstp = batch_pad.reshape(NS, POOL_NCHUNK, POOL_CHUNK)
    pooled = _pool_sc(
        ptab.reshape(POOL_SLABS * N, 128), srcp, dstp,
        jnp.zeros((POOL_RPS, POOL_D), f32),
    )

    mu, lv, dft = _latent(
        pooled, cnt2, eps,
        p["agg_W"], p["agg_b"].reshape(1, 256),
        p["mu_W"], p["mu_b"].reshape(1, 256),
        p["var_W"], p["var_b"].reshape(1, 256),
        p["dft_W"], p["dft_b"].reshape(1, 1536),
    )

    y2 = _dec_a(dft.reshape(N, 256), dis, p["d1_W"])
    (y2,) = _gcn_layer(y2, srcg, dstg, dis, p["d1_b"].reshape(1, 256),
                       wn=p["d2_W"])
    (y2,) = _gcn_layer(y2, srcg, dstg, dis, p["d2_b"].reshape(1, 256),
                       wn=p["d3_W"])
    accd3 = _gcn_sc(y2.reshape(2 * NP, 128), srcg, dstg).reshape(2, NP, 128)
    out = _heads(accd3, dis, p["d3_b"].reshape(1, 256), w0cat, b0cat, bdp, b1p)

    pos_out = out[:, 0:2]
    acttype = out[:, 2:8]
    direc = out[:, 8:9]
    laneidx = out[:, 9:19]
    return (pos_out, acttype, direc, laneidx, lv, mu)


# speedup vs baseline: 8.9486x; 1.0009x over previous
import functools

import jax
import jax.numpy as jnp
from jax import lax
from jax.experimental import pallas as pl
from jax.experimental.pallas import tpu as pltpu
from jax.experimental.pallas import tpu_sc as plsc

N = 12000
E = 192000
G = 2000
NP = 12288

NC, NS = 2, 16

_MESH = plsc.VectorSubcoreMesh(
    core_axis_name="c", subcore_axis_name="s", num_cores=NC, num_subcores=NS
)


ROWS_PER_SUB = NP // NS
EDGES_PER_TILE = E // NS
GCN_CHUNK = 60
GCN_BLK = 25
GCN_NBLK = 8


@functools.partial(
    pl.kernel,
    out_type=jax.ShapeDtypeStruct((2 * NP, 128), jnp.float32),
    mesh=_MESH,
    scratch_types=[
        pltpu.VMEM((GCN_BLK, GCN_CHUNK), jnp.int32),
        pltpu.VMEM((GCN_BLK, GCN_CHUNK), jnp.int32),
        pltpu.VMEM((3, GCN_CHUNK, 128), jnp.float32),
        pltpu.VMEM_SHARED((NP, 128), jnp.float32),
        pltpu.SemaphoreType.DMA((3,)),
        pltpu.SemaphoreType.DMA((3,)),
    ],
)
def _gcn_sc(table, src, dst, out, sidx2, didx2, rows2, acc, sem_g, sem_s):
    core = lax.axis_index("c")
    sub = lax.axis_index("s")
    pltpu.sync_copy(
        table.at[pl.ds(core * NP + sub * ROWS_PER_SUB, ROWS_PER_SUB)],
        acc.at[pl.ds(sub * ROWS_PER_SUB, ROWS_PER_SUB)],
    )
    plsc.subcore_barrier()

    def block(b, carry0):
        pltpu.sync_copy(src.at[core, sub, b], sidx2)
        pltpu.sync_copy(dst.at[sub, b], didx2)
        pltpu.async_copy(table.at[sidx2.at[0]], rows2.at[0], sem_g.at[0])
        pltpu.async_copy(table.at[sidx2.at[1]], rows2.at[1], sem_g.at[1])

        def step(k, carry):
            slot = lax.rem(k, 3)
            pltpu.make_async_copy(
                table.at[sidx2.at[k]], rows2.at[slot], sem_g.at[slot]
            ).wait()
            pltpu.async_copy(
                rows2.at[slot], acc.at[didx2.at[k]], sem_s.at[slot], add=True
            )

            @pl.when(k + 2 < GCN_BLK)
            def _fire_next_gather():
                nslot = lax.rem(k + 2, 3)

                @pl.when(k > 0)
                def _wait_prev_scatter():
                    pltpu.make_async_copy(
                        rows2.at[nslot], acc.at[didx2.at[k - 1]], sem_s.at[nslot]
                    ).wait()

                pltpu.async_copy(
                    table.at[sidx2.at[k + 2]], rows2.at[nslot], sem_g.at[nslot]
                )

            return carry

        lax.fori_loop(0, GCN_BLK, step, 0)
        for t in (GCN_BLK - 3, GCN_BLK - 2, GCN_BLK - 1):
            pltpu.make_async_copy(
                rows2.at[t % 3], acc.at[didx2.at[t]], sem_s.at[t % 3]
            ).wait()
        return carry0

    lax.fori_loop(0, GCN_NBLK, block, 0)
    plsc.subcore_barrier()
    pltpu.sync_copy(
        acc.at[pl.ds(sub * ROWS_PER_SUB, ROWS_PER_SUB)],
        out.at[pl.ds(core * NP + sub * ROWS_PER_SUB, ROWS_PER_SUB)],
    )


DEG_PER_TILE = (E // 2) // NS
DEG_RPS = NP // NS


CNT_BINS = 2048
CNT_PER_TILE = 12288 // 2 // NS
CNT_RPS = CNT_BINS // NS


@functools.partial(
    pl.kernel,
    out_type=(
        jax.ShapeDtypeStruct((2, NP), jnp.float32),
        jax.ShapeDtypeStruct((2, CNT_BINS), jnp.float32),
    ),
    mesh=_MESH,
    compiler_params=pltpu.CompilerParams(needs_layout_passes=False),
    scratch_types=[
        pltpu.VMEM((DEG_PER_TILE,), jnp.int32),
        pltpu.VMEM((CNT_PER_TILE,), jnp.int32),
        pltpu.VMEM((NP,), jnp.float32),
        pltpu.VMEM((CNT_BINS,), jnp.float32),
        pltpu.VMEM((NS, DEG_RPS), jnp.float32),
        pltpu.VMEM((DEG_RPS,), jnp.float32),
        pltpu.VMEM((NS, CNT_RPS), jnp.float32),
        pltpu.VMEM((CNT_RPS,), jnp.float32),
        pltpu.VMEM_SHARED((NS, NP), jnp.float32),
        pltpu.VMEM_SHARED((NS, CNT_BINS), jnp.float32),
    ],
)
def _deg_sc(dst, bat, out, outc, didx1, bidx1, hist, histc, colsum, acc1,
            colsumc, acc1c, shared, sharedc):
    core = lax.axis_index("c")
    sub = lax.axis_index("s")
    pltpu.sync_copy(
        dst.at[pl.ds(core * (E // 2) + sub * DEG_PER_TILE, DEG_PER_TILE)],
        didx1,
    )
    pltpu.sync_copy(
        bat.at[pl.ds(core * (12288 // 2) + sub * CNT_PER_TILE, CNT_PER_TILE)],
        bidx1,
    )
    zeros16 = jnp.zeros((16,), jnp.float32)
    ones16 = jnp.ones((16,), jnp.float32)

    def zstep(i, c):
        hist[pl.ds(i * 16, 16)] = zeros16
        return c

    lax.fori_loop(0, NP // 16, zstep, 0)

    def zstepc(i, c):
        histc[pl.ds(i * 16, 16)] = zeros16
        return c

    lax.fori_loop(0, CNT_BINS // 16, zstepc, 0)

    def hstep(k, c):
        v = didx1[pl.ds(k * 16, 16)]
        plsc.addupdate_scatter(hist, [v], ones16)
        return c

    lax.fori_loop(0, DEG_PER_TILE // 16, hstep, 0)

    def hstepc(k, c):
        v = bidx1[pl.ds(k * 16, 16)]
        plsc.addupdate_scatter(histc, [v], ones16)
        return c

    lax.fori_loop(0, CNT_PER_TILE // 16, hstepc, 0)
    pltpu.sync_copy(hist, shared.at[sub])
    pltpu.sync_copy(histc, sharedc.at[sub])
    plsc.subcore_barrier()
    pltpu.sync_copy(shared.at[:, pl.ds(sub * DEG_RPS, DEG_RPS)], colsum)

    def rstep(j, c):
        s = colsum[0, pl.ds(j * 16, 16)]
        for r in range(1, NS):
            s = s + colsum[r, pl.ds(j * 16, 16)]
        acc1[pl.ds(j * 16, 16)] = s
        return c

    lax.fori_loop(0, DEG_RPS // 16, rstep, 0)
    pltpu.sync_copy(acc1, out.at[core, pl.ds(sub * DEG_RPS, DEG_RPS)])
    pltpu.sync_copy(sharedc.at[:, pl.ds(sub * CNT_RPS, CNT_RPS)], colsumc)

    def rstepc(j, c):
        s = colsumc[0, pl.ds(j * 16, 16)]
        for r in range(1, NS):
            s = s + colsumc[r, pl.ds(j * 16, 16)]
        acc1c[pl.ds(j * 16, 16)] = s
        return c

    lax.fori_loop(0, CNT_RPS // 16, rstepc, 0)
    pltpu.sync_copy(acc1c, outc.at[core, pl.ds(sub * CNT_RPS, CNT_RPS)])


POOL_ROWS = 2048
POOL_ITEMS = 12288
POOL_PER_TILE = POOL_ITEMS // NS
POOL_CHUNK = 96
POOL_NCHUNK = POOL_PER_TILE // POOL_CHUNK
POOL_RPS = POOL_ROWS // NS
POOL_D = 128
POOL_SLABS = 10


@functools.partial(
    pl.kernel,
    out_type=jax.ShapeDtypeStruct((POOL_SLABS, POOL_ROWS, POOL_D), jnp.float32),
    mesh=_MESH,
    scratch_types=[
        pltpu.VMEM((POOL_NCHUNK, POOL_CHUNK), jnp.int32),
        pltpu.VMEM((POOL_NCHUNK, POOL_CHUNK), jnp.int32),
        pltpu.VMEM((2, POOL_CHUNK, POOL_D), jnp.float32),
        pltpu.VMEM_SHARED((POOL_ROWS, POOL_D), jnp.float32),
        pltpu.SemaphoreType.DMA((2,)),
        pltpu.SemaphoreType.DMA((2,)),
    ],
)
def _pool_sc(table, src, dst, zinit, out, sidx2, didx2, rows2, acc, sem_g, sem_s):
    core = lax.axis_index("c")
    sub = lax.axis_index("s")
    pltpu.sync_copy(dst.at[sub], didx2)

    def ppass(p, carry0):
        h = core * (POOL_SLABS // 2) + p
        pltpu.sync_copy(src.at[h, sub], sidx2)
        pltpu.sync_copy(zinit, acc.at[pl.ds(sub * POOL_RPS, POOL_RPS)])
        plsc.subcore_barrier()

        pltpu.async_copy(table.at[sidx2.at[0]], rows2.at[0], sem_g.at[0])

        def step(k, carry):
            slot = lax.rem(k, 2)
            nslot = lax.rem(k + 1, 2)

            @pl.when(k > 0)
            def _wait_prev_scatter():
                pltpu.make_async_copy(
                    rows2.at[nslot], acc.at[didx2.at[k - 1]], sem_s.at[nslot]
                ).wait()

            @pl.when(k + 1 < POOL_NCHUNK)
            def _fire_next_gather():
                pltpu.async_copy(
                    table.at[sidx2.at[k + 1]], rows2.at[nslot], sem_g.at[nslot]
                )

            pltpu.make_async_copy(
                table.at[sidx2.at[k]], rows2.at[slot], sem_g.at[slot]
            ).wait()
            pltpu.async_copy(
                rows2.at[slot], acc.at[didx2.at[k]], sem_s.at[slot], add=True
            )
            return carry

        lax.fori_loop(0, POOL_NCHUNK, step, 0)
        pltpu.make_async_copy(
            rows2.at[(POOL_NCHUNK - 1) % 2],
            acc.at[didx2.at[POOL_NCHUNK - 1]],
            sem_s.at[(POOL_NCHUNK - 1) % 2],
        ).wait()
        plsc.subcore_barrier()
        pltpu.sync_copy(
            acc.at[pl.ds(sub * POOL_RPS, POOL_RPS)],
            out.at[h, pl.ds(sub * POOL_RPS, POOL_RPS)],
        )
        plsc.subcore_barrier()
        return carry0

    lax.fori_loop(0, POOL_SLABS // 2, ppass, 0)


BM = 512
NB = (N + BM - 1) // BM
NBG = 4


def _finalize_deg(degfull):
    def body(d_ref, o_ref):
        s = d_ref[0] + d_ref[1] + 1.0
        o_ref[...] = jnp.broadcast_to(
            lax.rsqrt(s).reshape(NP, 1), (NP, 128)
        )

    return pl.pallas_call(
        body,
        out_shape=jax.ShapeDtypeStruct((NP, 128), jnp.float32),
        grid=(1,),
        in_specs=[pl.BlockSpec((2, NP), lambda i: (0, 0))],
        out_specs=pl.BlockSpec((NP, 128), lambda i: (0, 0)),
    )(degfull)


def _enc_a1(x19, dis, wenc, benc, w1):
    def body(x_ref, dis_ref, we_ref, be_ref, w1_ref, pt_ref, y2_ref):
        n0 = jnp.maximum(
            jnp.dot(x_ref[...], we_ref[...], preferred_element_type=jnp.float32)
            + be_ref[...],
            0.0,
        )
        for j in range(4):
            pt_ref[j] = n0[:, j * 128:(j + 1) * 128]
        d = dis_ref[:, 0:1]
        y = jnp.dot(n0, w1_ref[...], preferred_element_type=jnp.float32) * d
        y2_ref[0] = y[:, :128]
        y2_ref[1] = y[:, 128:]

    return pl.pallas_call(
        body,
        out_shape=(
            jax.ShapeDtypeStruct((POOL_SLABS, N, 128), jnp.float32),
            jax.ShapeDtypeStruct((2, NP, 128), jnp.float32),
        ),
        grid=(NB,),
        in_specs=[
            pl.BlockSpec((BM, 19), lambda i: (i, 0)),
            pl.BlockSpec((BM, 128), lambda i: (i, 0)),
            pl.BlockSpec((19, 512), lambda i: (0, 0)),
            pl.BlockSpec((1, 512), lambda i: (0, 0)),
            pl.BlockSpec((512, 256), lambda i: (0, 0)),
        ],
        out_specs=(
            pl.BlockSpec((4, BM, 128), lambda i: (0, i, 0)),
            pl.BlockSpec((2, BM, 128), lambda i: (0, i, 0)),
        ),
    )(x19, dis, wenc, benc, w1)


def _gcn_post(acc2, dis, b, wn=None, ptab=None, slab=0):
    def body(*refs):
        n_in = 3 + (wn is not None) + (ptab is not None)
        a_ref, dis_ref, b_ref = refs[:3]
        wn_ref = refs[3] if wn is not None else None
        orefs = refs[n_in:]
        d = dis_ref[:, 0:1]
        a = jnp.concatenate([a_ref[0], a_ref[1]], axis=1)
        h = jnp.maximum(a * d + b_ref[...], 0.0)
        i = 0
        if ptab is not None:
            orefs[i][0] = h[:, :128]
            orefs[i][1] = h[:, 128:]
            i += 1
        if wn is not None:
            y = jnp.dot(h, wn_ref[...], preferred_element_type=jnp.float32) * d
            orefs[i][0] = y[:, :128]
            orefs[i][1] = y[:, 128:]

    in_specs = [
        pl.BlockSpec((2, BM, 128), lambda i: (0, i, 0)),
        pl.BlockSpec((BM, 128), lambda i: (i, 0)),
        pl.BlockSpec((1, 256), lambda i: (0, 0)),
    ]
    args = [acc2, dis, b]
    if wn is not None:
        in_specs.append(pl.BlockSpec((256, 256), lambda i: (0, 0)))
        args.append(wn)
    aliases = {}
    out_shape, out_specs = [], []
    if ptab is not None:
        aliases[len(args)] = 0
        in_specs.append(pl.BlockSpec(memory_space=pl.ANY))
        args.append(ptab)
        out_shape.append(
            jax.ShapeDtypeStruct((POOL_SLABS, N, 128), jnp.float32)
        )
        sb = slab // 2
        out_specs.append(pl.BlockSpec((2, BM, 128), lambda i: (sb, i, 0)))
    if wn is not None:
        out_shape.append(jax.ShapeDtypeStruct((2, NP, 128), jnp.float32))
        out_specs.append(pl.BlockSpec((2, BM, 128), lambda i: (0, i, 0)))
    return pl.pallas_call(
        body,
        out_shape=tuple(out_shape),
        grid=(NB,),
        in_specs=in_specs,
        out_specs=tuple(out_specs),
        input_output_aliases=aliases,
    )(*args)


def _latent(pooled, cnt2, eps, aggw, aggb, muw, mub, varw, varb, dftw, dftb):
    def body(p_ref, c_ref, e_ref, aw, ab, mw, mb, vw, vb, dw, db,
             mu_ref, lv_ref, dft_ref):
        cnt = (c_ref[0] + c_ref[1]).reshape(c_ref.shape[1], 1)
        g = jnp.concatenate(
            [p_ref[h] for h in range(10)], axis=1
        ) / jnp.maximum(cnt, 1.0)
        latent = jnp.dot(g, aw[...], preferred_element_type=jnp.float32) + ab[...]
        mu = jnp.dot(latent, mw[...], preferred_element_type=jnp.float32) + mb[...]
        lv = jnp.dot(latent, vw[...], preferred_element_type=jnp.float32) + vb[...]
        mu_ref[...] = mu
        lv_ref[...] = lv
        z = e_ref[...] * jnp.exp(0.5 * lv) + mu
        dft_ref[...] = jnp.dot(z, dw[...], preferred_element_type=jnp.float32) + db[...]

    return pl.pallas_call(
        body,
        out_shape=(
            jax.ShapeDtypeStruct((G, 256), jnp.float32),
            jax.ShapeDtypeStruct((G, 256), jnp.float32),
            jax.ShapeDtypeStruct((G, 1536), jnp.float32),
        ),
        grid=(NBG,),
        in_specs=[
            pl.BlockSpec((POOL_SLABS, BM, POOL_D), lambda i: (0, i, 0)),
            pl.BlockSpec((2, BM), lambda i: (0, i)),
            pl.BlockSpec((BM, 256), lambda i: (i, 0)),
            pl.BlockSpec((1280, 256), lambda i: (0, 0)),
            pl.BlockSpec((1, 256), lambda i: (0, 0)),
            pl.BlockSpec((256, 256), lambda i: (0, 0)),
            pl.BlockSpec((1, 256), lambda i: (0, 0)),
            pl.BlockSpec((256, 256), lambda i: (0, 0)),
            pl.BlockSpec((1, 256), lambda i: (0, 0)),
            pl.BlockSpec((256, 1536), lambda i: (0, 0)),
            pl.BlockSpec((1, 1536), lambda i: (0, 0)),
        ],
        out_specs=(
            pl.BlockSpec((BM, 256), lambda i: (i, 0)),
            pl.BlockSpec((BM, 256), lambda i: (i, 0)),
            pl.BlockSpec((BM, 1536), lambda i: (i, 0)),
        ),
    )(pooled, cnt2, eps, aggw, aggb, muw, mub, varw, varb, dftw, dftb)


def _dec_a(z6, dis, w):
    def body(z_ref, dis_ref, w_ref, y2_ref):
        d = dis_ref[:, 0:1]
        h = jnp.maximum(z_ref[...], 0.0)
        y = jnp.dot(h, w_ref[...], preferred_element_type=jnp.float32) * d
        y2_ref[0] = y[:, :128]
        y2_ref[1] = y[:, 128:]

    return pl.pallas_call(
        body,
        out_shape=jax.ShapeDtypeStruct((2, NP, 128), jnp.float32),
        grid=(NB,),
        in_specs=[
            pl.BlockSpec((BM, 256), lambda i: (i, 0)),
            pl.BlockSpec((BM, 128), lambda i: (i, 0)),
            pl.BlockSpec((256, 256), lambda i: (0, 0)),
        ],
        out_specs=pl.BlockSpec((2, BM, 128), lambda i: (0, i, 0)),
    )(z6, dis, w)


def _heads(acc2, dis, b3, w0cat, b0cat, bdp, b1p):
    def body(a_ref, dis_ref, b3_ref, w0_ref, b0_ref, bd_ref, b1_ref, o_ref):
        d = dis_ref[:, 0:1]
        a = jnp.concatenate([a_ref[0], a_ref[1]], axis=1)
        h3 = jnp.maximum(a * d + b3_ref[...], 0.0)
        hh = jnp.maximum(
            jnp.dot(h3, w0_ref[...], preferred_element_type=jnp.float32)
            + b0_ref[...],
            0.0,
        )
        o_ref[...] = (
            jnp.dot(hh, bd_ref[...], preferred_element_type=jnp.float32)
            + b1_ref[...]
        )

    return pl.pallas_call(
        body,
        out_shape=jax.ShapeDtypeStruct((N, 128), jnp.float32),
        grid=(NB,),
        in_specs=[
            pl.BlockSpec((2, BM, 128), lambda i: (0, i, 0)),
            pl.BlockSpec((BM, 128), lambda i: (i, 0)),
            pl.BlockSpec((1, 256), lambda i: (0, 0)),
            pl.BlockSpec((256, 1280), lambda i: (0, 0)),
            pl.BlockSpec((1, 1280), lambda i: (0, 0)),
            pl.BlockSpec((1280, 128), lambda i: (0, 0)),
            pl.BlockSpec((1, 128), lambda i: (0, 0)),
        ],
        out_specs=pl.BlockSpec((BM, 128), lambda i: (i, 0)),
    )(acc2, dis, b3, w0cat, b0cat, bdp, b1p)


def _gcn_layer(y2, srcg, dstg, dis, b, wn=None, ptab=None, slab=0):
    acc = _gcn_sc(y2.reshape(2 * NP, 128), srcg, dstg)
    return _gcn_post(acc.reshape(2, NP, 128), dis, b, wn=wn, ptab=ptab,
                     slab=slab)


def kernel(pos, actor_type, lane_index, direction, params, edge_index, batch):
    p = params
    f32 = jnp.float32
    row, col = edge_index[0], edge_index[1]
    srcg = jnp.concatenate([row, row + NP]).reshape(
        2, NS, GCN_NBLK, GCN_BLK, GCN_CHUNK
    )
    dstg = col.reshape(NS, GCN_NBLK, GCN_BLK, GCN_CHUNK)

    wenc = jnp.zeros((19, 512), f32)
    wenc = wenc.at[0:2, 0:128].set(p["pos_W"])
    wenc = wenc.at[2:8, 128:256].set(p["type_W"])
    wenc = wenc.at[8:18, 256:384].set(p["lane_W"])
    wenc = wenc.at[18:19, 384:512].set(p["dir_W"])
    benc = jnp.concatenate(
        [p["pos_b"], p["type_b"], p["lane_b"], p["dir_b"]]
    ).reshape(1, 512)
    x19 = jnp.concatenate([pos, actor_type, lane_index, direction], axis=1)

    w0cat = jnp.concatenate(
        [p["px0_W"], p["py0_W"], p["at0_W"], p["dr0_W"], p["li0_W"]], axis=1
    )
    b0cat = jnp.concatenate(
        [p["px0_b"], p["py0_b"], p["at0_b"], p["dr0_b"], p["li0_b"]]
    ).reshape(1, 1280)
    bdp = jnp.zeros((1280, 128), f32)
    bdp = bdp.at[0:256, 0:1].set(p["px1_W"])
    bdp = bdp.at[256:512, 1:2].set(p["py1_W"])
    bdp = bdp.at[512:768, 2:8].set(p["at1_W"])
    bdp = bdp.at[768:1024, 8:9].set(p["dr1_W"])
    bdp = bdp.at[1024:1280, 9:19].set(p["li1_W"])
    b1p = jnp.zeros((128,), f32)
    b1p = b1p.at[0:1].set(p["px1_b"])
    b1p = b1p.at[1:2].set(p["py1_b"])
    b1p = b1p.at[2:8].set(p["at1_b"])
    b1p = b1p.at[8:9].set(p["dr1_b"])
    b1p = b1p.at[9:19].set(p["li1_b"])
    b1p = b1p.reshape(1, 128)

    eps = jax.random.normal(jax.random.key(42), (G, 256), dtype=f32)

    batch_pad = jnp.concatenate(
        [batch, jnp.full((POOL_ITEMS - N,), 2047, jnp.int32)]
    )

    degfull, cnt2 = _deg_sc(col, batch_pad)
    dis = _finalize_deg(degfull)

    ptab, y2 = _enc_a1(x19, dis, wenc, benc, p["e1_W"])
    (ptab, y2) = _gcn_layer(y2, srcg, dstg, dis, p["e1_b"].reshape(1, 256),
                            wn=p["e2_W"], ptab=ptab, slab=4)
    (ptab, y2) = _gcn_layer(y2, srcg, dstg, dis, p["e2_b"].reshape(1, 256),
                            wn=p["e3_W"], ptab=ptab, slab=6)
    (ptab,) = _gcn_layer(y2, srcg, dstg, dis, p["e3_b"].reshape(1, 256),
                         wn=None, ptab=ptab, slab=8)

    ar = jnp.arange(N, dtype=jnp.int32)
    pad_src = jnp.zeros((POOL_ITEMS - N,), jnp.int32)
    srcp_half = jnp.concatenate([ar, pad_src])
    srcp = jnp.concatenate(
        [srcp_half + h * N for h in range(POOL_SLABS)]
    ).reshape(POOL_SLABS, NS, POOL_NCHUNK, POOL_CHUNK)
    dstp = batch_pad.reshape(NS, POOL_NCHUNK, POOL_CHUNK)
    pooled = _pool_sc(
        ptab.reshape(POOL_SLABS * N, 128), srcp, dstp,
        jnp.zeros((POOL_RPS, POOL_D), f32),
    )

    mu, lv, dft = _latent(
        pooled, cnt2, eps,
        p["agg_W"], p["agg_b"].reshape(1, 256),
        p["mu_W"], p["mu_b"].reshape(1, 256),
        p["var_W"], p["var_b"].reshape(1, 256),
        p["dft_W"], p["dft_b"].reshape(1, 1536),
    )

    y2 = _dec_a(dft.reshape(N, 256), dis, p["d1_W"])
    (y2,) = _gcn_layer(y2, srcg, dstg, dis, p["d1_b"].reshape(1, 256),
                       wn=p["d2_W"])
    (y2,) = _gcn_layer(y2, srcg, dstg, dis, p["d2_b"].reshape(1, 256),
                       wn=p["d3_W"])
    accd3 = _gcn_sc(y2.reshape(2 * NP, 128), srcg, dstg).reshape(2, NP, 128)
    out = _heads(accd3, dis, p["d3_b"].reshape(1, 256), w0cat, b0cat, bdp, b1p)

    pos_out = out[:, 0:2]
    acttype = out[:, 2:8]
    direc = out[:, 8:9]
    laneidx = out[:, 9:19]
    return (pos_out, acttype, direc, laneidx, lv, mu)
